# baseline (speedup 1.0000x reference)
import functools

import jax
import jax.numpy as jnp
import numpy as np
from jax import lax
from jax.experimental import pallas as pl
from jax.experimental.pallas import tpu as pltpu
from jax.experimental.pallas import tpu_sc as plsc

C = 256
EBLK = 512
NC, NS = 2, 16
NW = NC * NS
GCHUNK = 128

_INTERPRET = False


def _cdiv(a, b):
    return (a + b - 1) // b


def _embed_call(x_pad, wyk8, wa1, t2, b_atom):
    n = x_pad.shape[0]

    def body(x_ref, wy_ref, wa_ref, t2_ref, b_ref, out_ref):
        x = x_ref[...]
        ids = wy_ref[...][:, 0:1]
        oh = (ids == lax.broadcasted_iota(jnp.int32, (1, 128), 1)).astype(jnp.float32)
        out_ref[...] = x @ wa_ref[...] + oh @ t2_ref[...] + b_ref[...]

    return pl.pallas_call(
        body,
        out_shape=jax.ShapeDtypeStruct((n, C), jnp.float32),
        interpret=_INTERPRET,
    )(x_pad, wyk8, wa1, t2, b_atom)


def _ef_call(ea8, wr1, u_vec, w_vec, b_rbf, n_real):
    epad = ea8.shape[0]
    grid = epad // EBLK
    gamma2 = float((255.0 / 4.0) ** 2)

    def body(ea_ref, wr_ref, u_ref, w_ref, b_ref, out_ref):
        ea = ea_ref[...]
        vx, vy, vz = ea[:, 0:1], ea[:, 1:2], ea[:, 2:3]
        nrm = jnp.sqrt(vx * vx + vy * vy + vz * vz)
        is_inv = ea[:, 3:4]
        ity = ea[:, 4:5]
        s = jnp.where(is_inv > 0.5, ity, -0.75 / nrm)
        cent = -4.0 + lax.broadcasted_iota(
            jnp.int32, (1, C), 1).astype(jnp.float32) * (4.0 / 255.0)
        r = jnp.exp(-gamma2 * (s - cent) ** 2)
        pre = r @ wr_ref[...] + is_inv * (ity * u_ref[...] + w_ref[...]) + b_ref[...]
        out_ref[...] = jax.nn.softplus(pre)

    return pl.pallas_call(
        body,
        grid=(grid,),
        in_specs=[
            pl.BlockSpec((EBLK, 8), lambda i: (i, 0)),
            pl.BlockSpec((C, C), lambda i: (0, 0)),
            pl.BlockSpec((1, C), lambda i: (0, 0)),
            pl.BlockSpec((1, C), lambda i: (0, 0)),
            pl.BlockSpec((1, C), lambda i: (0, 0)),
        ],
        out_specs=pl.BlockSpec((EBLK, C), lambda i: (i, 0)),
        out_shape=jax.ShapeDtypeStruct((epad, C), jnp.float32),
        interpret=_INTERPRET,
    )(ea8, wr1, u_vec, w_vec, b_rbf)


def _conv_edge_call(hds, ef, p, n_real):
    epad = ef.shape[0]
    grid = epad // EBLK
    nb = grid
    inv_s = float(1.0 / np.sqrt(3.0 * C))

    def body(hd_ref, hs_ref, ef_ref, wq_ref, wk_ref, wv_ref, we_ref,
             bq_ref, bk_ref, bv_ref, be_ref, wm1_ref, wm2_ref, wm3_ref,
             bm_ref, g1_ref, b1_ref, wmsg_ref, bmsg_ref, g2_ref, b2_ref,
             m0_ref, m1_ref):
        bf = jnp.bfloat16
        f32 = jnp.float32

        def dot16(a, b_ref):
            return jnp.dot(a.astype(bf), b_ref[...],
                           preferred_element_type=f32)

        hd = hd_ref[...]
        hs = hs_ref[...]
        ef_b = ef_ref[...]
        q_d = dot16(hd, wq_ref) + bq_ref[...]
        k_d = dot16(hd, wk_ref) + bk_ref[...]
        k_s = dot16(hs, wk_ref) + bk_ref[...]
        v_d = dot16(hd, wv_ref) + bv_ref[...]
        v_s = dot16(hs, wv_ref) + bv_ref[...]
        e = dot16(ef_b, we_ref) + be_ref[...]
        alpha = jnp.concatenate([q_d * k_d, q_d * k_s, q_d * e], axis=1) * inv_s
        mu = jnp.mean(alpha, axis=1, keepdims=True)
        var = jnp.mean(alpha * alpha, axis=1, keepdims=True) - mu * mu
        gate = jax.nn.sigmoid((alpha - mu) * lax.rsqrt(var + 1e-5)
                              * g1_ref[...] + b1_ref[...])
        m = (dot16(v_d, wm1_ref) + dot16(v_s, wm2_ref) + dot16(e, wm3_ref)
             + bm_ref[...])
        m = m * gate
        msg = dot16(m, wmsg_ref) + bmsg_ref[...]
        mu2 = jnp.mean(msg, axis=1, keepdims=True)
        var2 = jnp.mean(msg * msg, axis=1, keepdims=True) - mu2 * mu2
        msg = (msg - mu2) * lax.rsqrt(var2 + 1e-5) * g2_ref[...] + b2_ref[...]
        eid = pl.program_id(0) * EBLK + lax.broadcasted_iota(jnp.int32, (EBLK, 1), 0)
        msg = jnp.where(eid < n_real, msg, 0.0)
        m0_ref[...] = msg[:, :128]
        m1_ref[...] = msg[:, 128:]

    w_spec = lambda shape: pl.BlockSpec(shape, lambda i: (0, 0))
    out = pl.pallas_call(
        body,
        grid=(grid,),
        in_specs=[
            pl.BlockSpec((EBLK, C), lambda i: (i, 0)),
            pl.BlockSpec((EBLK, C), lambda i: (i + nb, 0)),
            pl.BlockSpec((EBLK, C), lambda i: (i, 0)),
            w_spec((C, C)), w_spec((C, C)), w_spec((C, C)), w_spec((C, C)),
            w_spec((1, C)), w_spec((1, C)), w_spec((1, C)), w_spec((1, C)),
            w_spec((C, 3 * C)), w_spec((C, 3 * C)), w_spec((C, 3 * C)),
            w_spec((1, 3 * C)), w_spec((1, 3 * C)), w_spec((1, 3 * C)),
            w_spec((3 * C, C)), w_spec((1, C)), w_spec((1, C)), w_spec((1, C)),
        ],
        out_specs=[
            pl.BlockSpec((EBLK, 128), lambda i: (i, 0)),
            pl.BlockSpec((EBLK, 128), lambda i: (i, 0)),
        ],
        out_shape=[
            jax.ShapeDtypeStruct((epad, 128), jnp.float32),
            jax.ShapeDtypeStruct((epad, 128), jnp.float32),
        ],
        interpret=_INTERPRET,
    )
    bf = jnp.bfloat16
    out = out(hds, hds, ef,
              p["Wq"].astype(bf), p["Wk"].astype(bf), p["Wv"].astype(bf),
              p["We"].astype(bf),
              p["bq"].reshape(1, C), p["bk"].reshape(1, C),
              p["bv"].reshape(1, C), p["be"].reshape(1, C),
              p["Wm"][:C].astype(bf), p["Wm"][C:2 * C].astype(bf),
              p["Wm"][2 * C:].astype(bf),
              p["bm"].reshape(1, 3 * C), p["g1"].reshape(1, 3 * C),
              p["b1"].reshape(1, 3 * C),
              p["Wmsg"].astype(bf), p["bmsg"].reshape(1, C),
              p["g2"].reshape(1, C), p["b2"].reshape(1, C))
    return out


def _conv_node_call(agg0, agg1, h, p):
    n = h.shape[0]

    def body(a0_ref, a1_ref, h_ref, wc0_ref, wc1_ref, bc_ref, g_ref, b_ref,
             out_ref):
        out = a0_ref[...] @ wc0_ref[...] + a1_ref[...] @ wc1_ref[...] + bc_ref[...]
        mu = jnp.mean(out, axis=0, keepdims=True)
        var = jnp.mean(out * out, axis=0, keepdims=True) - mu * mu
        bn = (out - mu) * lax.rsqrt(var + 1e-5) * g_ref[...] + b_ref[...]
        out_ref[...] = jax.nn.softplus(h_ref[...] + bn)

    return pl.pallas_call(
        body,
        grid=(2,),
        in_specs=[
            pl.BlockSpec((n, 128), lambda j: (0, 0)),
            pl.BlockSpec((n, 128), lambda j: (0, 0)),
            pl.BlockSpec((n, 128), lambda j: (0, j)),
            pl.BlockSpec((128, 128), lambda j: (0, j)),
            pl.BlockSpec((128, 128), lambda j: (0, j)),
            pl.BlockSpec((1, 128), lambda j: (0, j)),
            pl.BlockSpec((1, 128), lambda j: (0, j)),
            pl.BlockSpec((1, 128), lambda j: (0, j)),
        ],
        out_specs=pl.BlockSpec((n, 128), lambda j: (0, j)),
        out_shape=jax.ShapeDtypeStruct((n, C), jnp.float32),
        interpret=_INTERPRET,
    )(agg0, agg1, h, p["Wc"][:128], p["Wc"][128:], p["bc"].reshape(1, C),
      p["gbn"].reshape(1, C), p["bbn"].reshape(1, C))


def _hn_call(h, wn_pad):
    n = h.shape[0]

    def body(h_ref, w_ref, out_ref):
        out_ref[...] = h_ref[...] @ w_ref[...]

    return pl.pallas_call(
        body,
        out_shape=jax.ShapeDtypeStruct((n, 128), jnp.float32),
        interpret=_INTERPRET,
    )(h, wn_pad)


def _equi_edge_call(hns, ef, ea8, wes, bes, n_real):
    epad = ef.shape[0]
    grid = epad // EBLK
    c0, c1, c2, c3, c4 = (0.28209479177, 0.48860251190, 1.09254843059,
                          0.31539156525, 0.54627421529)

    def body(hn_ref, ef_ref, ea_ref, wes_ref, bes_ref, *out_refs):
        ea = ea_ref[...]
        vx, vy, vz = ea[:, 0:1], ea[:, 1:2], ea[:, 2:3]
        nrm = jnp.sqrt(vx * vx + vy * vy + vz * vz) + 1e-8
        x = vx / nrm
        y = vy / nrm
        z = vz / nrm
        es = jax.nn.silu(ef_ref[...] @ wes_ref[...] + bes_ref[...])
        m = hn_ref[...][:, :32] * es
        eid = pl.program_id(0) * EBLK + lax.broadcasted_iota(jnp.int32, (EBLK, 1), 0)
        m = jnp.where(eid < n_real, m, 0.0)
        sh = [jnp.full_like(x, c0), c1 * y, c1 * z, c1 * x,
              c2 * x * y, c2 * y * z, c3 * (3.0 * z * z - 1.0),
              c2 * x * z, c4 * (x * x - y * y)]
        out_refs[0][...] = jnp.concatenate([m * sh[j] for j in range(4)], axis=1)
        out_refs[1][...] = jnp.concatenate([m * sh[j] for j in range(4, 8)], axis=1)
        out_refs[2][...] = jnp.concatenate(
            [m * sh[8], jnp.zeros((EBLK, 96), jnp.float32)], axis=1)

    return pl.pallas_call(
        body,
        grid=(grid,),
        in_specs=[
            pl.BlockSpec((EBLK, 128), lambda i: (i, 0)),
            pl.BlockSpec((EBLK, C), lambda i: (i, 0)),
            pl.BlockSpec((EBLK, 8), lambda i: (i, 0)),
            pl.BlockSpec((C, 32), lambda i: (0, 0)),
            pl.BlockSpec((1, 32), lambda i: (0, 0)),
        ],
        out_specs=[pl.BlockSpec((EBLK, 128), lambda i: (i, 0))] * 3,
        out_shape=[jax.ShapeDtypeStruct((epad, 128), jnp.float32)] * 3,
        interpret=_INTERPRET,
    )(hns, ef, ea8, wes, bes)


def _equi_node_call(h, aggs, wtps, gln, bln):
    n = h.shape[0]

    def body(*refs):
        h_ref = refs[0]
        agg_refs = refs[1:4]
        wtp_refs = refs[4:7]
        g_ref, b_ref = refs[7], refs[8]
        out_ref = refs[9]
        acc = agg_refs[0][...] @ wtp_refs[0][...]
        for j in range(1, 3):
            acc = acc + agg_refs[j][...] @ wtp_refs[j][...]
        mu = jnp.mean(acc, axis=1, keepdims=True)
        var = jnp.mean(acc * acc, axis=1, keepdims=True) - mu * mu
        ln = (acc - mu) * lax.rsqrt(var + 1e-5) * g_ref[...] + b_ref[...]
        out_ref[...] = h_ref[...] + jax.nn.silu(ln)

    rblk = 1000 if n % 1000 == 0 else n
    return pl.pallas_call(
        body,
        grid=(n // rblk,),
        in_specs=[pl.BlockSpec((rblk, C), lambda i: (i, 0))]
        + [pl.BlockSpec((rblk, 128), lambda i: (i, 0))] * 3
        + [pl.BlockSpec((128, C), lambda i: (0, 0))] * 3
        + [pl.BlockSpec((1, C), lambda i: (0, 0))] * 2,
        out_specs=pl.BlockSpec((rblk, C), lambda i: (i, 0)),
        out_shape=jax.ShapeDtypeStruct((n, C), jnp.float32),
        interpret=_INTERPRET,
    )(h, *aggs, *wtps, gln.reshape(1, C), bln.reshape(1, C))


def _pool_call(h, batch8, wfc, bfc, wout_pad, bout_pad, ng):
    n = h.shape[0]

    def body(h_ref, b_ref, wfc_ref, bfc_ref, wo_ref, bo_ref, out_ref):
        ids = b_ref[...][:, 0:1]
        oh = (ids == lax.broadcasted_iota(jnp.int32, (1, ng), 1)).astype(jnp.float32)
        pooled = lax.dot_general(oh, h_ref[...], (((0,), (0,)), ((), ())))
        ones = jnp.ones((n, 1), jnp.float32)
        cnt = lax.dot_general(oh, ones, (((0,), (0,)), ((), ())))
        pooled = pooled / jnp.maximum(cnt, 1.0)
        hh = jax.nn.silu(pooled @ wfc_ref[...] + bfc_ref[...])
        logits = hh @ wo_ref[...] + bo_ref[...]
        l4 = logits[:, 0:4]
        mx = jnp.max(l4, axis=1, keepdims=True)
        lse = jnp.log(jnp.sum(jnp.exp(l4 - mx), axis=1, keepdims=True))
        res = l4 - mx - lse
        pad = jnp.zeros((ng, 124), jnp.float32)
        out_ref[...] = jnp.concatenate([res, pad], axis=1)

    return pl.pallas_call(
        body,
        out_shape=jax.ShapeDtypeStruct((ng, 128), jnp.float32),
        interpret=_INTERPRET,
    )(h, batch8, wfc, bfc, wout_pad, bout_pad)


def _gather_rows(table, idx, width):
    n_rows = idx.shape[0]
    per_tile = n_rows // NW
    n_chunks = per_tile // GCHUNK
    mesh = plsc.VectorSubcoreMesh(core_axis_name="c", subcore_axis_name="s")

    @functools.partial(
        pl.kernel,
        mesh=mesh,
        out_type=jax.ShapeDtypeStruct((n_rows, width), jnp.float32),
        scratch_types=[
            pltpu.VMEM((per_tile,), jnp.int32),
            pltpu.VMEM((GCHUNK, width), jnp.float32),
            pltpu.VMEM((GCHUNK, width), jnp.float32),
            pltpu.SemaphoreType.DMA,
            pltpu.SemaphoreType.DMA,
            pltpu.SemaphoreType.DMA,
            pltpu.SemaphoreType.DMA,
        ],
    )
    def k(table_hbm, idx_hbm, out_hbm, idx_all, buf0, buf1,
          sg0, sg1, sw0, sw1):
        cid = lax.axis_index("c")
        sid = lax.axis_index("s")
        base = pl.multiple_of((sid * NC + cid) * per_tile, GCHUNK)
        pltpu.sync_copy(idx_hbm.at[pl.ds(base, per_tile)], idx_all)

        def gstart(j, buf, sem):
            o = pl.multiple_of(j * GCHUNK, GCHUNK)
            return pltpu.async_copy(
                table_hbm.at[idx_all.at[pl.ds(o, GCHUNK)]], buf, sem)

        def wstart(j, buf, sem):
            r = pl.multiple_of(base + j * GCHUNK, GCHUNK)
            return pltpu.async_copy(buf, out_hbm.at[pl.ds(r, GCHUNK)], sem)

        gstart(0, buf0, sg0)

        def body(jj, carry):
            j0 = jj * 2
            o = pl.multiple_of(j0 * GCHUNK, GCHUNK)
            pltpu.make_async_copy(
                table_hbm.at[idx_all.at[pl.ds(o, GCHUNK)]], buf0, sg0).wait()
            w0 = wstart(j0, buf0, sw0)
            g1 = gstart(j0 + 1, buf1, sg1)
            g1.wait()
            w1 = wstart(j0 + 1, buf1, sw1)
            w0.wait()

            @pl.when(jj < n_chunks // 2 - 1)
            def _():
                gstart(j0 + 2, buf0, sg0)

            w1.wait()
            return carry

        lax.fori_loop(0, n_chunks // 2, body, 0)

    return k(table, idx)


def _scatter_add(msgs, dst, n_nodes, core_of):
    epad = dst.shape[0]
    per_tile = epad // NW
    n_chunks = per_tile // GCHUNK
    stripe = (n_nodes // NS) // 8 * 8
    last_stripe = n_nodes - (NS - 1) * stripe
    na = len(msgs)
    widths = [m.shape[1] for m in msgs]
    w = widths[0]
    assert all(wi == w for wi in widths)
    groups = [[a for a in range(na) if core_of[a] == c] for c in range(NC)]
    assert max(len(g) for g in groups) == 1
    mesh = plsc.VectorSubcoreMesh(core_axis_name="c", subcore_axis_name="s")

    zeros = jnp.zeros((last_stripe, w), jnp.float32)
    dst3 = dst.reshape(NW, n_chunks, GCHUNK)

    scratch = [
        pltpu.VMEM((n_chunks, GCHUNK), jnp.int32),
        pltpu.VMEM((GCHUNK, w), jnp.float32),
        pltpu.VMEM((GCHUNK, w), jnp.float32),
        pltpu.VMEM_SHARED((n_nodes, w), jnp.float32),
        pltpu.SemaphoreType.DMA,
        pltpu.SemaphoreType.DMA,
    ]

    @functools.partial(
        pl.kernel,
        mesh=mesh,
        out_type=[jax.ShapeDtypeStruct((n_nodes, w), jnp.float32)
                  for _ in range(na)],
        scratch_types=scratch,
    )
    def k(*refs):
        msg_refs = refs[0:na]
        dst_ref = refs[na]
        zero_ref = refs[na + 1]
        out_refs = refs[na + 2:2 * na + 2]
        idx2d, buf0, buf1, acc, sl0, sl1 = refs[2 * na + 2:]

        cid = lax.axis_index("c")
        sid = lax.axis_index("s")
        wid = sid * NC + cid
        base = pl.multiple_of(wid * per_tile, GCHUNK)
        off = pl.multiple_of(sid * stripe, 8)

        for length, pred in ((stripe, sid < NS - 1),
                             (last_stripe, sid == NS - 1)):

            @pl.when(pred)
            def _(length=length):
                pltpu.sync_copy(zero_ref.at[pl.ds(0, length)],
                                acc.at[pl.ds(off, length)])

        plsc.subcore_barrier()

        for c in range(NC):
            if not groups[c]:
                continue
            a = groups[c][0]

            @pl.when(cid == c)
            def _(a=a):
                mref = msg_refs[a]
                pltpu.sync_copy(dst_ref.at[wid], idx2d)

                def lstart(j, buf, sem):
                    r = pl.multiple_of(base + j * GCHUNK, GCHUNK)
                    return pltpu.async_copy(mref.at[pl.ds(r, GCHUNK)], buf, sem)

                lstart(0, buf0, sl0)

                def body(jj, carry):
                    j0 = jj * 2
                    r0 = pl.multiple_of(base + j0 * GCHUNK, GCHUNK)
                    pltpu.make_async_copy(
                        mref.at[pl.ds(r0, GCHUNK)], buf0, sl0).wait()
                    g1 = lstart(j0 + 1, buf1, sl1)
                    pltpu.sync_copy(buf0, acc.at[idx2d.at[j0]], add=True)
                    g1.wait()

                    @pl.when(jj < n_chunks // 2 - 1)
                    def _():
                        lstart(j0 + 2, buf0, sl0)

                    pltpu.sync_copy(buf1, acc.at[idx2d.at[j0 + 1]], add=True)
                    return carry

                lax.fori_loop(0, n_chunks // 2, body, 0)

        plsc.subcore_barrier()

        for c in range(NC):
            if not groups[c]:
                continue
            a = groups[c][0]
            for length, pred in ((stripe, sid < NS - 1),
                                 (last_stripe, sid == NS - 1)):

                @pl.when(jnp.logical_and(cid == c, pred))
                def _(a=a, length=length):
                    pltpu.sync_copy(acc.at[pl.ds(off, length)],
                                    out_refs[a].at[pl.ds(off, length)])

    return k(*msgs, dst3, zeros)


def kernel(x, edge_attr, inv_edge_attr, params, wyckoff, edge_index,
           inv_edge_index, inv_edge_type, batch):
    n = x.shape[0]
    e_r = edge_attr.shape[0]
    e_i = inv_edge_attr.shape[0]
    ne = e_r + e_i
    epad = _cdiv(ne, NW * GCHUNK * 2) * NW * GCHUNK * 2
    ng = 64
    f32 = jnp.float32

    ea8 = jnp.concatenate([
        jnp.concatenate([edge_attr, jnp.zeros((e_r, 5), f32)], axis=1),
        jnp.concatenate([inv_edge_attr, jnp.ones((e_i, 1), f32),
                         inv_edge_type.astype(f32)[:, None],
                         jnp.zeros((e_i, 3), f32)], axis=1),
        jnp.concatenate([jnp.ones((epad - ne, 1), f32),
                         jnp.zeros((epad - ne, 7), f32)], axis=1),
    ], axis=0)

    zpad = jnp.zeros((epad - ne,), jnp.int32)
    src = jnp.concatenate([edge_index[0], inv_edge_index[0], zpad])
    dst = jnp.concatenate([edge_index[1], inv_edge_index[1], zpad])
    idx_ds = jnp.concatenate([dst, src])

    x_pad = jnp.concatenate([x, jnp.zeros((n, 128 - x.shape[1]), f32)], axis=1)
    wyk8 = jnp.broadcast_to(wyckoff.astype(jnp.int32)[:, None], (n, 8))
    batch8 = jnp.broadcast_to(batch.astype(jnp.int32)[:, None], (n, 8))

    wa1 = jnp.concatenate([params["W_atom"][:x.shape[1]],
                           jnp.zeros((128 - x.shape[1], C), f32)], axis=0)
    t2 = params["wyckoff_table"] @ params["W_atom"][x.shape[1]:]
    t2 = jnp.concatenate([t2, jnp.zeros((128 - t2.shape[0], C), f32)], axis=0)

    w2 = params["W_rbf"][C:]
    u_vec = (params["W_inv"] @ w2).reshape(1, C)
    w_vec = (params["b_inv"] @ w2).reshape(1, C)

    equi = params["equi"]
    wtp = equi["Wtp"]
    wtp_g = [
        jnp.concatenate([wtp[j::9] for j in range(4)], axis=0),
        jnp.concatenate([wtp[j::9] for j in range(4, 8)], axis=0),
        jnp.concatenate([wtp[8::9], jnp.zeros((96, C), f32)], axis=0),
    ]
    wn_pad = jnp.concatenate([equi["Wn"], jnp.zeros((C, 96), f32)], axis=1)

    wout_pad = jnp.concatenate([params["W_out"],
                                jnp.zeros((C, 124), f32)], axis=1)
    bout_pad = jnp.concatenate([params["b_out"],
                                jnp.zeros((124,), f32)]).reshape(1, 128)

    h = _embed_call(x_pad, wyk8, wa1, t2, params["b_atom"].reshape(1, C))
    ef = _ef_call(ea8, params["W_rbf"][:C], u_vec, w_vec,
                  params["b_rbf"].reshape(1, C), ne)

    def conv(h, p):
        h16 = lax.bitcast_convert_type(
            h.astype(jnp.bfloat16).reshape(n, 128, 2), f32)
        hds16 = _gather_rows(h16, idx_ds, 128)
        hds = lax.bitcast_convert_type(hds16, jnp.bfloat16).reshape(-1, C)
        msg0, msg1 = _conv_edge_call(hds, ef, p, ne)
        agg0, agg1 = _scatter_add([msg0, msg1], dst, n, [0, 1])
        return _conv_node_call(agg0, agg1, h, p)

    h = conv(h, params["conv0"])

    hn = _hn_call(h, wn_pad)
    hns = _gather_rows(hn, src, 128)
    tps = _equi_edge_call(hns, ef, ea8, equi["Wes"],
                          equi["bes"].reshape(1, 32), ne)
    aggs01 = _scatter_add(tps[:2], dst, n, [0, 1])
    aggs2 = _scatter_add(tps[2:], dst, n, [0])
    h = _equi_node_call(h, aggs01 + aggs2, wtp_g, equi["gln"], equi["bln"])

    h = conv(h, params["conv1"])
    h = conv(h, params["conv2"])

    out = _pool_call(h, batch8, params["W_fc"], params["b_fc"].reshape(1, C),
                     wout_pad, bout_pad, ng)
    return out[:, :4]


# speedup vs baseline: 1.8682x; 1.8682x over previous
import functools

import jax
import jax.numpy as jnp
import numpy as np
from jax import lax
from jax.experimental import pallas as pl
from jax.experimental.pallas import tpu as pltpu
from jax.experimental.pallas import tpu_sc as plsc

C = 256
EBLK = 512
NC, NS = 2, 16
NW = NC * NS
GCHUNK = 128

_INTERPRET = False


def _cdiv(a, b):
    return (a + b - 1) // b


def _embed_call(x_pad, wyk8, wa1, t2, b_atom):
    n = x_pad.shape[0]

    def body(x_ref, wy_ref, wa_ref, t2_ref, b_ref, out_ref):
        x = x_ref[...]
        ids = wy_ref[...][:, 0:1]
        oh = (ids == lax.broadcasted_iota(jnp.int32, (1, 128), 1)).astype(jnp.float32)
        out_ref[...] = x @ wa_ref[...] + oh @ t2_ref[...] + b_ref[...]

    return pl.pallas_call(
        body,
        out_shape=jax.ShapeDtypeStruct((n, C), jnp.float32),
        interpret=_INTERPRET,
    )(x_pad, wyk8, wa1, t2, b_atom)


def _ef_call(ea8, wr1, u_vec, w_vec, b_rbf, n_real):
    epad = ea8.shape[0]
    grid = epad // EBLK
    gamma2 = float((255.0 / 4.0) ** 2)

    def body(ea_ref, wr_ref, u_ref, w_ref, b_ref, out_ref):
        ea = ea_ref[...]
        vx, vy, vz = ea[:, 0:1], ea[:, 1:2], ea[:, 2:3]
        nrm = jnp.sqrt(vx * vx + vy * vy + vz * vz)
        is_inv = ea[:, 3:4]
        ity = ea[:, 4:5]
        s = jnp.where(is_inv > 0.5, ity, -0.75 / nrm)
        cent = -4.0 + lax.broadcasted_iota(
            jnp.int32, (1, C), 1).astype(jnp.float32) * (4.0 / 255.0)
        r = jnp.exp(-gamma2 * (s - cent) ** 2)
        pre = r @ wr_ref[...] + is_inv * (ity * u_ref[...] + w_ref[...]) + b_ref[...]
        out_ref[...] = jax.nn.softplus(pre)

    return pl.pallas_call(
        body,
        grid=(grid,),
        in_specs=[
            pl.BlockSpec((EBLK, 8), lambda i: (i, 0)),
            pl.BlockSpec((C, C), lambda i: (0, 0)),
            pl.BlockSpec((1, C), lambda i: (0, 0)),
            pl.BlockSpec((1, C), lambda i: (0, 0)),
            pl.BlockSpec((1, C), lambda i: (0, 0)),
        ],
        out_specs=pl.BlockSpec((EBLK, C), lambda i: (i, 0)),
        out_shape=jax.ShapeDtypeStruct((epad, C), jnp.float32),
        interpret=_INTERPRET,
    )(ea8, wr1, u_vec, w_vec, b_rbf)


def _pack_call(h):
    n = h.shape[0]

    def body(h_ref, out_ref):
        hv = h_ref[...]
        a = lax.bitcast_convert_type(
            hv[:, :128].astype(jnp.bfloat16), jnp.uint16).astype(jnp.uint32)
        b = lax.bitcast_convert_type(
            hv[:, 128:].astype(jnp.bfloat16), jnp.uint16).astype(jnp.uint32)
        out_ref[...] = lax.bitcast_convert_type(a | (b << 16), jnp.float32)

    return pl.pallas_call(
        body,
        out_shape=jax.ShapeDtypeStruct((n, 128), jnp.float32),
        interpret=_INTERPRET,
    )(h)


def _conv_edge_call(hds, ef, p, n_real):
    epad = ef.shape[0]
    grid = epad // EBLK
    nb = grid
    inv_s = float(1.0 / np.sqrt(3.0 * C))

    def body(hd_ref, hs_ref, ef_ref, wq_ref, wk_ref, wv_ref, we_ref,
             bq_ref, bk_ref, bv_ref, be_ref, wm1_ref, wm2_ref, wm3_ref,
             bm_ref, g1_ref, b1_ref, wmsg_ref, bmsg_ref, g2_ref, b2_ref,
             m0_ref, m1_ref):
        bf = jnp.bfloat16
        f32 = jnp.float32

        def dot16(a, b_ref):
            return jnp.dot(a.astype(bf), b_ref[...],
                           preferred_element_type=f32)

        def unpack(ref):
            u = lax.bitcast_convert_type(ref[...], jnp.uint32)
            lo = lax.bitcast_convert_type(
                (u & 0xFFFF).astype(jnp.uint16), bf)
            hi = lax.bitcast_convert_type(
                (u >> 16).astype(jnp.uint16), bf)
            return lo, hi

        def dot2(lohi, w_ref):
            w = w_ref[...]
            return (jnp.dot(lohi[0], w[:128], preferred_element_type=f32)
                    + jnp.dot(lohi[1], w[128:], preferred_element_type=f32))

        hd = unpack(hd_ref)
        hs = unpack(hs_ref)
        ef_b = ef_ref[...]
        q_d = dot2(hd, wq_ref) + bq_ref[...]
        k_d = dot2(hd, wk_ref) + bk_ref[...]
        k_s = dot2(hs, wk_ref) + bk_ref[...]
        v_d = dot2(hd, wv_ref) + bv_ref[...]
        v_s = dot2(hs, wv_ref) + bv_ref[...]
        e = dot16(ef_b, we_ref) + be_ref[...]
        alpha = jnp.concatenate([q_d * k_d, q_d * k_s, q_d * e], axis=1) * inv_s
        mu = jnp.mean(alpha, axis=1, keepdims=True)
        var = jnp.mean(alpha * alpha, axis=1, keepdims=True) - mu * mu
        gate = jax.nn.sigmoid((alpha - mu) * lax.rsqrt(var + 1e-5)
                              * g1_ref[...] + b1_ref[...])
        m = (dot16(v_d, wm1_ref) + dot16(v_s, wm2_ref) + dot16(e, wm3_ref)
             + bm_ref[...])
        m = m * gate
        msg = dot16(m, wmsg_ref) + bmsg_ref[...]
        mu2 = jnp.mean(msg, axis=1, keepdims=True)
        var2 = jnp.mean(msg * msg, axis=1, keepdims=True) - mu2 * mu2
        msg = (msg - mu2) * lax.rsqrt(var2 + 1e-5) * g2_ref[...] + b2_ref[...]
        eid = pl.program_id(0) * EBLK + lax.broadcasted_iota(jnp.int32, (EBLK, 1), 0)
        msg = jnp.where(eid < n_real, msg, 0.0)
        m0_ref[...] = msg[:, :128]
        m1_ref[...] = msg[:, 128:]

    w_spec = lambda shape: pl.BlockSpec(shape, lambda i: (0, 0))
    out = pl.pallas_call(
        body,
        grid=(grid,),
        in_specs=[
            pl.BlockSpec((EBLK, 128), lambda i: (i, 0)),
            pl.BlockSpec((EBLK, 128), lambda i: (i + nb, 0)),
            pl.BlockSpec((EBLK, C), lambda i: (i, 0)),
            w_spec((C, C)), w_spec((C, C)), w_spec((C, C)), w_spec((C, C)),
            w_spec((1, C)), w_spec((1, C)), w_spec((1, C)), w_spec((1, C)),
            w_spec((C, 3 * C)), w_spec((C, 3 * C)), w_spec((C, 3 * C)),
            w_spec((1, 3 * C)), w_spec((1, 3 * C)), w_spec((1, 3 * C)),
            w_spec((3 * C, C)), w_spec((1, C)), w_spec((1, C)), w_spec((1, C)),
        ],
        out_specs=[
            pl.BlockSpec((EBLK, 128), lambda i: (i, 0)),
            pl.BlockSpec((EBLK, 128), lambda i: (i, 0)),
        ],
        out_shape=[
            jax.ShapeDtypeStruct((epad, 128), jnp.float32),
            jax.ShapeDtypeStruct((epad, 128), jnp.float32),
        ],
        interpret=_INTERPRET,
    )
    bf = jnp.bfloat16
    out = out(hds, hds, ef,
              p["Wq"].astype(bf), p["Wk"].astype(bf), p["Wv"].astype(bf),
              p["We"].astype(bf),
              p["bq"].reshape(1, C), p["bk"].reshape(1, C),
              p["bv"].reshape(1, C), p["be"].reshape(1, C),
              p["Wm"][:C].astype(bf), p["Wm"][C:2 * C].astype(bf),
              p["Wm"][2 * C:].astype(bf),
              p["bm"].reshape(1, 3 * C), p["g1"].reshape(1, 3 * C),
              p["b1"].reshape(1, 3 * C),
              p["Wmsg"].astype(bf), p["bmsg"].reshape(1, C),
              p["g2"].reshape(1, C), p["b2"].reshape(1, C))
    return out


def _conv_node_call(agg0, agg1, h, p):
    n = h.shape[0]

    def body(a0_ref, a1_ref, h_ref, wc0_ref, wc1_ref, bc_ref, g_ref, b_ref,
             out_ref):
        out = a0_ref[...] @ wc0_ref[...] + a1_ref[...] @ wc1_ref[...] + bc_ref[...]
        mu = jnp.mean(out, axis=0, keepdims=True)
        var = jnp.mean(out * out, axis=0, keepdims=True) - mu * mu
        bn = (out - mu) * lax.rsqrt(var + 1e-5) * g_ref[...] + b_ref[...]
        out_ref[...] = jax.nn.softplus(h_ref[...] + bn)

    return pl.pallas_call(
        body,
        grid=(2,),
        in_specs=[
            pl.BlockSpec((n, 128), lambda j: (0, 0)),
            pl.BlockSpec((n, 128), lambda j: (0, 0)),
            pl.BlockSpec((n, 128), lambda j: (0, j)),
            pl.BlockSpec((128, 128), lambda j: (0, j)),
            pl.BlockSpec((128, 128), lambda j: (0, j)),
            pl.BlockSpec((1, 128), lambda j: (0, j)),
            pl.BlockSpec((1, 128), lambda j: (0, j)),
            pl.BlockSpec((1, 128), lambda j: (0, j)),
        ],
        out_specs=pl.BlockSpec((n, 128), lambda j: (0, j)),
        out_shape=jax.ShapeDtypeStruct((n, C), jnp.float32),
        interpret=_INTERPRET,
    )(agg0, agg1, h, p["Wc"][:128], p["Wc"][128:], p["bc"].reshape(1, C),
      p["gbn"].reshape(1, C), p["bbn"].reshape(1, C))


def _hn_call(h, wn_pad):
    n = h.shape[0]

    def body(h_ref, w_ref, out_ref):
        out_ref[...] = h_ref[...] @ w_ref[...]

    return pl.pallas_call(
        body,
        out_shape=jax.ShapeDtypeStruct((n, 128), jnp.float32),
        interpret=_INTERPRET,
    )(h, wn_pad)


def _equi_edge_call(hns, ef, ea8, wes, bes, n_real):
    epad = ef.shape[0]
    grid = epad // EBLK
    c0, c1, c2, c3, c4 = (0.28209479177, 0.48860251190, 1.09254843059,
                          0.31539156525, 0.54627421529)

    def body(hn_ref, ef_ref, ea_ref, wes_ref, bes_ref, *out_refs):
        ea = ea_ref[...]
        vx, vy, vz = ea[:, 0:1], ea[:, 1:2], ea[:, 2:3]
        nrm = jnp.sqrt(vx * vx + vy * vy + vz * vz) + 1e-8
        x = vx / nrm
        y = vy / nrm
        z = vz / nrm
        es = jax.nn.silu(ef_ref[...] @ wes_ref[...] + bes_ref[...])
        m = hn_ref[...][:, :32] * es
        eid = pl.program_id(0) * EBLK + lax.broadcasted_iota(jnp.int32, (EBLK, 1), 0)
        m = jnp.where(eid < n_real, m, 0.0)
        sh = [jnp.full_like(x, c0), c1 * y, c1 * z, c1 * x,
              c2 * x * y, c2 * y * z, c3 * (3.0 * z * z - 1.0),
              c2 * x * z, c4 * (x * x - y * y)]
        out_refs[0][...] = jnp.concatenate([m * sh[j] for j in range(4)], axis=1)
        out_refs[1][...] = jnp.concatenate([m * sh[j] for j in range(4, 8)], axis=1)
        out_refs[2][...] = jnp.concatenate(
            [m * sh[8], jnp.zeros((EBLK, 96), jnp.float32)], axis=1)

    return pl.pallas_call(
        body,
        grid=(grid,),
        in_specs=[
            pl.BlockSpec((EBLK, 128), lambda i: (i, 0)),
            pl.BlockSpec((EBLK, C), lambda i: (i, 0)),
            pl.BlockSpec((EBLK, 8), lambda i: (i, 0)),
            pl.BlockSpec((C, 32), lambda i: (0, 0)),
            pl.BlockSpec((1, 32), lambda i: (0, 0)),
        ],
        out_specs=[pl.BlockSpec((EBLK, 128), lambda i: (i, 0))] * 3,
        out_shape=[jax.ShapeDtypeStruct((epad, 128), jnp.float32)] * 3,
        interpret=_INTERPRET,
    )(hns, ef, ea8, wes, bes)


def _equi_node_call(h, aggs, wtps, gln, bln):
    n = h.shape[0]

    def body(*refs):
        h_ref = refs[0]
        agg_refs = refs[1:4]
        wtp_refs = refs[4:7]
        g_ref, b_ref = refs[7], refs[8]
        out_ref = refs[9]
        acc = agg_refs[0][...] @ wtp_refs[0][...]
        for j in range(1, 3):
            acc = acc + agg_refs[j][...] @ wtp_refs[j][...]
        mu = jnp.mean(acc, axis=1, keepdims=True)
        var = jnp.mean(acc * acc, axis=1, keepdims=True) - mu * mu
        ln = (acc - mu) * lax.rsqrt(var + 1e-5) * g_ref[...] + b_ref[...]
        out_ref[...] = h_ref[...] + jax.nn.silu(ln)

    rblk = 1000 if n % 1000 == 0 else n
    return pl.pallas_call(
        body,
        grid=(n // rblk,),
        in_specs=[pl.BlockSpec((rblk, C), lambda i: (i, 0))]
        + [pl.BlockSpec((rblk, 128), lambda i: (i, 0))] * 3
        + [pl.BlockSpec((128, C), lambda i: (0, 0))] * 3
        + [pl.BlockSpec((1, C), lambda i: (0, 0))] * 2,
        out_specs=pl.BlockSpec((rblk, C), lambda i: (i, 0)),
        out_shape=jax.ShapeDtypeStruct((n, C), jnp.float32),
        interpret=_INTERPRET,
    )(h, *aggs, *wtps, gln.reshape(1, C), bln.reshape(1, C))


def _pool_call(h, batch8, wfc, bfc, wout_pad, bout_pad, ng):
    n = h.shape[0]

    def body(h_ref, b_ref, wfc_ref, bfc_ref, wo_ref, bo_ref, out_ref):
        ids = b_ref[...][:, 0:1]
        oh = (ids == lax.broadcasted_iota(jnp.int32, (1, ng), 1)).astype(jnp.float32)
        pooled = lax.dot_general(oh, h_ref[...], (((0,), (0,)), ((), ())))
        ones = jnp.ones((n, 1), jnp.float32)
        cnt = lax.dot_general(oh, ones, (((0,), (0,)), ((), ())))
        pooled = pooled / jnp.maximum(cnt, 1.0)
        hh = jax.nn.silu(pooled @ wfc_ref[...] + bfc_ref[...])
        logits = hh @ wo_ref[...] + bo_ref[...]
        l4 = logits[:, 0:4]
        mx = jnp.max(l4, axis=1, keepdims=True)
        lse = jnp.log(jnp.sum(jnp.exp(l4 - mx), axis=1, keepdims=True))
        res = l4 - mx - lse
        pad = jnp.zeros((ng, 124), jnp.float32)
        out_ref[...] = jnp.concatenate([res, pad], axis=1)

    return pl.pallas_call(
        body,
        out_shape=jax.ShapeDtypeStruct((ng, 128), jnp.float32),
        interpret=_INTERPRET,
    )(h, batch8, wfc, bfc, wout_pad, bout_pad)


def _gather_rows(table, idx, width):
    n_rows = idx.shape[0]
    per_tile = n_rows // NW
    n_chunks = per_tile // GCHUNK
    mesh = plsc.VectorSubcoreMesh(core_axis_name="c", subcore_axis_name="s")

    @functools.partial(
        pl.kernel,
        mesh=mesh,
        out_type=jax.ShapeDtypeStruct((n_rows, width), jnp.float32),
        scratch_types=[
            pltpu.VMEM((per_tile,), jnp.int32),
            pltpu.VMEM((GCHUNK, width), jnp.float32),
            pltpu.VMEM((GCHUNK, width), jnp.float32),
            pltpu.SemaphoreType.DMA,
            pltpu.SemaphoreType.DMA,
            pltpu.SemaphoreType.DMA,
            pltpu.SemaphoreType.DMA,
        ],
    )
    def k(table_hbm, idx_hbm, out_hbm, idx_all, buf0, buf1,
          sg0, sg1, sw0, sw1):
        cid = lax.axis_index("c")
        sid = lax.axis_index("s")
        base = pl.multiple_of((sid * NC + cid) * per_tile, GCHUNK)
        pltpu.sync_copy(idx_hbm.at[pl.ds(base, per_tile)], idx_all)

        def gstart(j, buf, sem):
            o = pl.multiple_of(j * GCHUNK, GCHUNK)
            return pltpu.async_copy(
                table_hbm.at[idx_all.at[pl.ds(o, GCHUNK)]], buf, sem)

        def wstart(j, buf, sem):
            r = pl.multiple_of(base + j * GCHUNK, GCHUNK)
            return pltpu.async_copy(buf, out_hbm.at[pl.ds(r, GCHUNK)], sem)

        gstart(0, buf0, sg0)

        def body(jj, carry):
            j0 = jj * 2
            o = pl.multiple_of(j0 * GCHUNK, GCHUNK)
            pltpu.make_async_copy(
                table_hbm.at[idx_all.at[pl.ds(o, GCHUNK)]], buf0, sg0).wait()
            w0 = wstart(j0, buf0, sw0)
            g1 = gstart(j0 + 1, buf1, sg1)
            g1.wait()
            w1 = wstart(j0 + 1, buf1, sw1)
            w0.wait()

            @pl.when(jj < n_chunks // 2 - 1)
            def _():
                gstart(j0 + 2, buf0, sg0)

            w1.wait()
            return carry

        lax.fori_loop(0, n_chunks // 2, body, 0)

    return k(table, idx)


def _scatter_add(msgs, dst, n_nodes, core_of):
    epad = dst.shape[0]
    per_tile = epad // NW
    n_chunks = per_tile // GCHUNK
    stripe = (n_nodes // NS) // 8 * 8
    last_stripe = n_nodes - (NS - 1) * stripe
    na = len(msgs)
    widths = [m.shape[1] for m in msgs]
    w = widths[0]
    assert all(wi == w for wi in widths)
    groups = [[a for a in range(na) if core_of[a] == c] for c in range(NC)]
    assert max(len(g) for g in groups) == 1
    mesh = plsc.VectorSubcoreMesh(core_axis_name="c", subcore_axis_name="s")

    zeros = jnp.zeros((last_stripe, w), jnp.float32)
    dst3 = dst.reshape(NW, n_chunks, GCHUNK)

    scratch = [
        pltpu.VMEM((n_chunks, GCHUNK), jnp.int32),
        pltpu.VMEM((GCHUNK, w), jnp.float32),
        pltpu.VMEM((GCHUNK, w), jnp.float32),
        pltpu.VMEM_SHARED((n_nodes, w), jnp.float32),
        pltpu.SemaphoreType.DMA,
        pltpu.SemaphoreType.DMA,
    ]

    @functools.partial(
        pl.kernel,
        mesh=mesh,
        out_type=[jax.ShapeDtypeStruct((n_nodes, w), jnp.float32)
                  for _ in range(na)],
        scratch_types=scratch,
    )
    def k(*refs):
        msg_refs = refs[0:na]
        dst_ref = refs[na]
        zero_ref = refs[na + 1]
        out_refs = refs[na + 2:2 * na + 2]
        idx2d, buf0, buf1, acc, sl0, sl1 = refs[2 * na + 2:]

        cid = lax.axis_index("c")
        sid = lax.axis_index("s")
        wid = sid * NC + cid
        base = pl.multiple_of(wid * per_tile, GCHUNK)
        off = pl.multiple_of(sid * stripe, 8)

        for length, pred in ((stripe, sid < NS - 1),
                             (last_stripe, sid == NS - 1)):

            @pl.when(pred)
            def _(length=length):
                pltpu.sync_copy(zero_ref.at[pl.ds(0, length)],
                                acc.at[pl.ds(off, length)])

        plsc.subcore_barrier()

        for c in range(NC):
            if not groups[c]:
                continue
            a = groups[c][0]

            @pl.when(cid == c)
            def _(a=a):
                mref = msg_refs[a]
                pltpu.sync_copy(dst_ref.at[wid], idx2d)

                def lstart(j, buf, sem):
                    r = pl.multiple_of(base + j * GCHUNK, GCHUNK)
                    return pltpu.async_copy(mref.at[pl.ds(r, GCHUNK)], buf, sem)

                lstart(0, buf0, sl0)

                def body(jj, carry):
                    j0 = jj * 2
                    r0 = pl.multiple_of(base + j0 * GCHUNK, GCHUNK)
                    pltpu.make_async_copy(
                        mref.at[pl.ds(r0, GCHUNK)], buf0, sl0).wait()
                    g1 = lstart(j0 + 1, buf1, sl1)
                    pltpu.sync_copy(buf0, acc.at[idx2d.at[j0]], add=True)
                    g1.wait()

                    @pl.when(jj < n_chunks // 2 - 1)
                    def _():
                        lstart(j0 + 2, buf0, sl0)

                    pltpu.sync_copy(buf1, acc.at[idx2d.at[j0 + 1]], add=True)
                    return carry

                lax.fori_loop(0, n_chunks // 2, body, 0)

        plsc.subcore_barrier()

        for c in range(NC):
            if not groups[c]:
                continue
            a = groups[c][0]
            for length, pred in ((stripe, sid < NS - 1),
                                 (last_stripe, sid == NS - 1)):

                @pl.when(jnp.logical_and(cid == c, pred))
                def _(a=a, length=length):
                    pltpu.sync_copy(acc.at[pl.ds(off, length)],
                                    out_refs[a].at[pl.ds(off, length)])

    return k(*msgs, dst3, zeros)


def kernel(x, edge_attr, inv_edge_attr, params, wyckoff, edge_index,
           inv_edge_index, inv_edge_type, batch):
    n = x.shape[0]
    e_r = edge_attr.shape[0]
    e_i = inv_edge_attr.shape[0]
    ne = e_r + e_i
    epad = _cdiv(ne, NW * GCHUNK * 2) * NW * GCHUNK * 2
    ng = 64
    f32 = jnp.float32

    ea8 = jnp.concatenate([
        jnp.concatenate([edge_attr, jnp.zeros((e_r, 5), f32)], axis=1),
        jnp.concatenate([inv_edge_attr, jnp.ones((e_i, 1), f32),
                         inv_edge_type.astype(f32)[:, None],
                         jnp.zeros((e_i, 3), f32)], axis=1),
        jnp.concatenate([jnp.ones((epad - ne, 1), f32),
                         jnp.zeros((epad - ne, 7), f32)], axis=1),
    ], axis=0)

    zpad = jnp.zeros((epad - ne,), jnp.int32)
    src = jnp.concatenate([edge_index[0], inv_edge_index[0], zpad])
    dst = jnp.concatenate([edge_index[1], inv_edge_index[1], zpad])
    idx_ds = jnp.concatenate([dst, src])

    x_pad = jnp.concatenate([x, jnp.zeros((n, 128 - x.shape[1]), f32)], axis=1)
    wyk8 = jnp.broadcast_to(wyckoff.astype(jnp.int32)[:, None], (n, 8))
    batch8 = jnp.broadcast_to(batch.astype(jnp.int32)[:, None], (n, 8))

    wa1 = jnp.concatenate([params["W_atom"][:x.shape[1]],
                           jnp.zeros((128 - x.shape[1], C), f32)], axis=0)
    t2 = params["wyckoff_table"] @ params["W_atom"][x.shape[1]:]
    t2 = jnp.concatenate([t2, jnp.zeros((128 - t2.shape[0], C), f32)], axis=0)

    w2 = params["W_rbf"][C:]
    u_vec = (params["W_inv"] @ w2).reshape(1, C)
    w_vec = (params["b_inv"] @ w2).reshape(1, C)

    equi = params["equi"]
    wtp = equi["Wtp"]
    wtp_g = [
        jnp.concatenate([wtp[j::9] for j in range(4)], axis=0),
        jnp.concatenate([wtp[j::9] for j in range(4, 8)], axis=0),
        jnp.concatenate([wtp[8::9], jnp.zeros((96, C), f32)], axis=0),
    ]
    wn_pad = jnp.concatenate([equi["Wn"], jnp.zeros((C, 96), f32)], axis=1)

    wout_pad = jnp.concatenate([params["W_out"],
                                jnp.zeros((C, 124), f32)], axis=1)
    bout_pad = jnp.concatenate([params["b_out"],
                                jnp.zeros((124,), f32)]).reshape(1, 128)

    h = _embed_call(x_pad, wyk8, wa1, t2, params["b_atom"].reshape(1, C))
    ef = _ef_call(ea8, params["W_rbf"][:C], u_vec, w_vec,
                  params["b_rbf"].reshape(1, C), ne)

    def conv(h, p):
        hds = _gather_rows(_pack_call(h), idx_ds, 128)
        msg0, msg1 = _conv_edge_call(hds, ef, p, ne)
        agg0, agg1 = _scatter_add([msg0, msg1], dst, n, [0, 1])
        return _conv_node_call(agg0, agg1, h, p)

    h = conv(h, params["conv0"])

    hn = _hn_call(h, wn_pad)
    hns = _gather_rows(hn, src, 128)
    tps = _equi_edge_call(hns, ef, ea8, equi["Wes"],
                          equi["bes"].reshape(1, 32), ne)
    aggs01 = _scatter_add(tps[:2], dst, n, [0, 1])
    aggs2 = _scatter_add(tps[2:], dst, n, [0])
    h = _equi_node_call(h, aggs01 + aggs2, wtp_g, equi["gln"], equi["bln"])

    h = conv(h, params["conv1"])
    h = conv(h, params["conv2"])

    out = _pool_call(h, batch8, params["W_fc"], params["b_fc"].reshape(1, C),
                     wout_pad, bout_pad, ng)
    return out[:, :4]


# speedup vs baseline: 1.9279x; 1.0319x over previous
import functools

import jax
import jax.numpy as jnp
import numpy as np
from jax import lax
from jax.experimental import pallas as pl
from jax.experimental.pallas import tpu as pltpu
from jax.experimental.pallas import tpu_sc as plsc

C = 256
EBLK = 512
NC, NS = 2, 16
NW = NC * NS
GCHUNK = 128

_INTERPRET = False


def _cdiv(a, b):
    return (a + b - 1) // b


def _embed_call(x_pad, wyk8, wa1, t2, b_atom):
    n = x_pad.shape[0]

    def body(x_ref, wy_ref, wa_ref, t2_ref, b_ref, out_ref):
        x = x_ref[...]
        ids = wy_ref[...][:, 0:1]
        oh = (ids == lax.broadcasted_iota(jnp.int32, (1, 128), 1)).astype(jnp.float32)
        out_ref[...] = x @ wa_ref[...] + oh @ t2_ref[...] + b_ref[...]

    return pl.pallas_call(
        body,
        out_shape=jax.ShapeDtypeStruct((n, C), jnp.float32),
        interpret=_INTERPRET,
    )(x_pad, wyk8, wa1, t2, b_atom)


def _ef_call(ea8, wr1, u_vec, w_vec, b_rbf, n_real):
    epad = ea8.shape[0]
    grid = epad // EBLK
    gamma2 = float((255.0 / 4.0) ** 2)

    def body(ea_ref, wr_ref, u_ref, w_ref, b_ref, out_ref):
        ea = ea_ref[...]
        vx, vy, vz = ea[:, 0:1], ea[:, 1:2], ea[:, 2:3]
        nrm = jnp.sqrt(vx * vx + vy * vy + vz * vz)
        is_inv = ea[:, 3:4]
        ity = ea[:, 4:5]
        s = jnp.where(is_inv > 0.5, ity, -0.75 / nrm)
        cent = -4.0 + lax.broadcasted_iota(
            jnp.int32, (1, C), 1).astype(jnp.float32) * (4.0 / 255.0)
        r = jnp.exp(-gamma2 * (s - cent) ** 2)
        pre = r @ wr_ref[...] + is_inv * (ity * u_ref[...] + w_ref[...]) + b_ref[...]
        out_ref[...] = jax.nn.softplus(pre)

    return pl.pallas_call(
        body,
        grid=(grid,),
        in_specs=[
            pl.BlockSpec((EBLK, 8), lambda i: (i, 0)),
            pl.BlockSpec((C, C), lambda i: (0, 0)),
            pl.BlockSpec((1, C), lambda i: (0, 0)),
            pl.BlockSpec((1, C), lambda i: (0, 0)),
            pl.BlockSpec((1, C), lambda i: (0, 0)),
        ],
        out_specs=pl.BlockSpec((EBLK, C), lambda i: (i, 0)),
        out_shape=jax.ShapeDtypeStruct((epad, C), jnp.float32),
        interpret=_INTERPRET,
    )(ea8, wr1, u_vec, w_vec, b_rbf)


def _pack_call(h):
    n = h.shape[0]

    def body(h_ref, out_ref):
        hv = h_ref[...]
        a = lax.bitcast_convert_type(
            hv[:, :128].astype(jnp.bfloat16), jnp.uint16).astype(jnp.uint32)
        b = lax.bitcast_convert_type(
            hv[:, 128:].astype(jnp.bfloat16), jnp.uint16).astype(jnp.uint32)
        out_ref[...] = lax.bitcast_convert_type(a | (b << 16), jnp.float32)

    return pl.pallas_call(
        body,
        out_shape=jax.ShapeDtypeStruct((n, 128), jnp.float32),
        interpret=_INTERPRET,
    )(h)


def _conv_edge_call(hds, ef, p, n_real):
    epad = ef.shape[0]
    grid = epad // EBLK
    nb = grid
    inv_s = float(1.0 / np.sqrt(3.0 * C))

    def body(hd_ref, hs_ref, ef_ref, wq_ref, wk_ref, wv_ref, we_ref,
             bq_ref, bk_ref, bv_ref, be_ref, wm1_ref, wm2_ref, wm3_ref,
             bm_ref, g1_ref, b1_ref, wmsg_ref, bmsg_ref, g2_ref, b2_ref,
             m0_ref, m1_ref):
        bf = jnp.bfloat16
        f32 = jnp.float32

        def dot16(a, b_ref):
            return jnp.dot(a.astype(bf), b_ref[...],
                           preferred_element_type=f32)

        def unpack(ref):
            u = lax.bitcast_convert_type(ref[...], jnp.uint32)
            lo = lax.bitcast_convert_type(
                (u & 0xFFFF).astype(jnp.uint16), bf)
            hi = lax.bitcast_convert_type(
                (u >> 16).astype(jnp.uint16), bf)
            return lo, hi

        def dot2(lohi, w_ref):
            w = w_ref[...]
            return (jnp.dot(lohi[0], w[:128], preferred_element_type=f32)
                    + jnp.dot(lohi[1], w[128:], preferred_element_type=f32))

        hd = unpack(hd_ref)
        hs = unpack(hs_ref)
        ef_b = ef_ref[...]
        q_d = dot2(hd, wq_ref) + bq_ref[...]
        k_d = dot2(hd, wk_ref) + bk_ref[...]
        k_s = dot2(hs, wk_ref) + bk_ref[...]
        v_d = dot2(hd, wv_ref) + bv_ref[...]
        v_s = dot2(hs, wv_ref) + bv_ref[...]
        e = dot16(ef_b, we_ref) + be_ref[...]
        alpha = jnp.concatenate([q_d * k_d, q_d * k_s, q_d * e], axis=1) * inv_s
        mu = jnp.mean(alpha, axis=1, keepdims=True)
        var = jnp.mean(alpha * alpha, axis=1, keepdims=True) - mu * mu
        gate = jax.nn.sigmoid((alpha - mu) * lax.rsqrt(var + 1e-5)
                              * g1_ref[...] + b1_ref[...])
        m = (dot16(v_d, wm1_ref) + dot16(v_s, wm2_ref) + dot16(e, wm3_ref)
             + bm_ref[...])
        m = m * gate
        msg = dot16(m, wmsg_ref) + bmsg_ref[...]
        mu2 = jnp.mean(msg, axis=1, keepdims=True)
        var2 = jnp.mean(msg * msg, axis=1, keepdims=True) - mu2 * mu2
        msg = (msg - mu2) * lax.rsqrt(var2 + 1e-5) * g2_ref[...] + b2_ref[...]
        eid = pl.program_id(0) * EBLK + lax.broadcasted_iota(jnp.int32, (EBLK, 1), 0)
        msg = jnp.where(eid < n_real, msg, 0.0)
        m0_ref[...] = msg[:, :128]
        m1_ref[...] = msg[:, 128:]

    w_spec = lambda shape: pl.BlockSpec(shape, lambda i: (0, 0))
    out = pl.pallas_call(
        body,
        grid=(grid,),
        in_specs=[
            pl.BlockSpec((EBLK, 128), lambda i: (i, 0)),
            pl.BlockSpec((EBLK, 128), lambda i: (i + nb, 0)),
            pl.BlockSpec((EBLK, C), lambda i: (i, 0)),
            w_spec((C, C)), w_spec((C, C)), w_spec((C, C)), w_spec((C, C)),
            w_spec((1, C)), w_spec((1, C)), w_spec((1, C)), w_spec((1, C)),
            w_spec((C, 3 * C)), w_spec((C, 3 * C)), w_spec((C, 3 * C)),
            w_spec((1, 3 * C)), w_spec((1, 3 * C)), w_spec((1, 3 * C)),
            w_spec((3 * C, C)), w_spec((1, C)), w_spec((1, C)), w_spec((1, C)),
        ],
        out_specs=[
            pl.BlockSpec((EBLK, 128), lambda i: (i, 0)),
            pl.BlockSpec((EBLK, 128), lambda i: (i, 0)),
        ],
        out_shape=[
            jax.ShapeDtypeStruct((epad, 128), jnp.float32),
            jax.ShapeDtypeStruct((epad, 128), jnp.float32),
        ],
        interpret=_INTERPRET,
    )
    bf = jnp.bfloat16
    out = out(hds, hds, ef,
              p["Wq"].astype(bf), p["Wk"].astype(bf), p["Wv"].astype(bf),
              p["We"].astype(bf),
              p["bq"].reshape(1, C), p["bk"].reshape(1, C),
              p["bv"].reshape(1, C), p["be"].reshape(1, C),
              p["Wm"][:C].astype(bf), p["Wm"][C:2 * C].astype(bf),
              p["Wm"][2 * C:].astype(bf),
              p["bm"].reshape(1, 3 * C), p["g1"].reshape(1, 3 * C),
              p["b1"].reshape(1, 3 * C),
              p["Wmsg"].astype(bf), p["bmsg"].reshape(1, C),
              p["g2"].reshape(1, C), p["b2"].reshape(1, C))
    return out


def _conv_node_call(agg0, agg1, h, p):
    n = h.shape[0]

    def body(a0_ref, a1_ref, h_ref, wc0_ref, wc1_ref, bc_ref, g_ref, b_ref,
             out_ref):
        out = a0_ref[...] @ wc0_ref[...] + a1_ref[...] @ wc1_ref[...] + bc_ref[...]
        mu = jnp.mean(out, axis=0, keepdims=True)
        var = jnp.mean(out * out, axis=0, keepdims=True) - mu * mu
        bn = (out - mu) * lax.rsqrt(var + 1e-5) * g_ref[...] + b_ref[...]
        out_ref[...] = jax.nn.softplus(h_ref[...] + bn)

    return pl.pallas_call(
        body,
        grid=(2,),
        in_specs=[
            pl.BlockSpec((n, 128), lambda j: (0, 0)),
            pl.BlockSpec((n, 128), lambda j: (0, 0)),
            pl.BlockSpec((n, 128), lambda j: (0, j)),
            pl.BlockSpec((128, 128), lambda j: (0, j)),
            pl.BlockSpec((128, 128), lambda j: (0, j)),
            pl.BlockSpec((1, 128), lambda j: (0, j)),
            pl.BlockSpec((1, 128), lambda j: (0, j)),
            pl.BlockSpec((1, 128), lambda j: (0, j)),
        ],
        out_specs=pl.BlockSpec((n, 128), lambda j: (0, j)),
        out_shape=jax.ShapeDtypeStruct((n, C), jnp.float32),
        interpret=_INTERPRET,
    )(agg0, agg1, h, p["Wc"][:128], p["Wc"][128:], p["bc"].reshape(1, C),
      p["gbn"].reshape(1, C), p["bbn"].reshape(1, C))


def _hn_call(h, wn_pad):
    n = h.shape[0]

    def body(h_ref, w_ref, out_ref):
        out_ref[...] = h_ref[...] @ w_ref[...]

    return pl.pallas_call(
        body,
        out_shape=jax.ShapeDtypeStruct((n, 128), jnp.float32),
        interpret=_INTERPRET,
    )(h, wn_pad)


def _equi_edge_call(hns, ef, ea8, wes, bes, n_real):
    epad = ef.shape[0]
    grid = epad // EBLK
    c0, c1, c2, c3, c4 = (0.28209479177, 0.48860251190, 1.09254843059,
                          0.31539156525, 0.54627421529)

    def body(hn_ref, ef_ref, ea_ref, wes_ref, bes_ref, *out_refs):
        ea = ea_ref[...]
        vx, vy, vz = ea[:, 0:1], ea[:, 1:2], ea[:, 2:3]
        nrm = jnp.sqrt(vx * vx + vy * vy + vz * vz) + 1e-8
        x = vx / nrm
        y = vy / nrm
        z = vz / nrm
        es = jax.nn.silu(ef_ref[...] @ wes_ref[...] + bes_ref[...])
        m = hn_ref[...][:, :32] * es
        eid = pl.program_id(0) * EBLK + lax.broadcasted_iota(jnp.int32, (EBLK, 1), 0)
        m = jnp.where(eid < n_real, m, 0.0)
        sh = [jnp.full_like(x, c0), c1 * y, c1 * z, c1 * x,
              c2 * x * y, c2 * y * z, c3 * (3.0 * z * z - 1.0),
              c2 * x * z, c4 * (x * x - y * y)]
        out_refs[0][...] = jnp.concatenate([m * sh[j] for j in range(4)], axis=1)
        out_refs[1][...] = jnp.concatenate([m * sh[j] for j in range(4, 8)], axis=1)
        out_refs[2][...] = jnp.concatenate(
            [m * sh[8], jnp.zeros((EBLK, 96), jnp.float32)], axis=1)

    return pl.pallas_call(
        body,
        grid=(grid,),
        in_specs=[
            pl.BlockSpec((EBLK, 128), lambda i: (i, 0)),
            pl.BlockSpec((EBLK, C), lambda i: (i, 0)),
            pl.BlockSpec((EBLK, 8), lambda i: (i, 0)),
            pl.BlockSpec((C, 32), lambda i: (0, 0)),
            pl.BlockSpec((1, 32), lambda i: (0, 0)),
        ],
        out_specs=[pl.BlockSpec((EBLK, 128), lambda i: (i, 0))] * 3,
        out_shape=[jax.ShapeDtypeStruct((epad, 128), jnp.float32)] * 3,
        interpret=_INTERPRET,
    )(hns, ef, ea8, wes, bes)


def _equi_node_call(h, aggs, wtps, gln, bln):
    n = h.shape[0]

    def body(*refs):
        h_ref = refs[0]
        agg_refs = refs[1:4]
        wtp_refs = refs[4:7]
        g_ref, b_ref = refs[7], refs[8]
        out_ref = refs[9]
        acc = agg_refs[0][...] @ wtp_refs[0][...]
        for j in range(1, 3):
            acc = acc + agg_refs[j][...] @ wtp_refs[j][...]
        mu = jnp.mean(acc, axis=1, keepdims=True)
        var = jnp.mean(acc * acc, axis=1, keepdims=True) - mu * mu
        ln = (acc - mu) * lax.rsqrt(var + 1e-5) * g_ref[...] + b_ref[...]
        out_ref[...] = h_ref[...] + jax.nn.silu(ln)

    rblk = 1000 if n % 1000 == 0 else n
    return pl.pallas_call(
        body,
        grid=(n // rblk,),
        in_specs=[pl.BlockSpec((rblk, C), lambda i: (i, 0))]
        + [pl.BlockSpec((rblk, 128), lambda i: (i, 0))] * 3
        + [pl.BlockSpec((128, C), lambda i: (0, 0))] * 3
        + [pl.BlockSpec((1, C), lambda i: (0, 0))] * 2,
        out_specs=pl.BlockSpec((rblk, C), lambda i: (i, 0)),
        out_shape=jax.ShapeDtypeStruct((n, C), jnp.float32),
        interpret=_INTERPRET,
    )(h, *aggs, *wtps, gln.reshape(1, C), bln.reshape(1, C))


def _pool_call(h, batch8, wfc, bfc, wout_pad, bout_pad, ng):
    n = h.shape[0]

    def body(h_ref, b_ref, wfc_ref, bfc_ref, wo_ref, bo_ref, out_ref):
        ids = b_ref[...][:, 0:1]
        oh = (ids == lax.broadcasted_iota(jnp.int32, (1, ng), 1)).astype(jnp.float32)
        pooled = lax.dot_general(oh, h_ref[...], (((0,), (0,)), ((), ())))
        ones = jnp.ones((n, 1), jnp.float32)
        cnt = lax.dot_general(oh, ones, (((0,), (0,)), ((), ())))
        pooled = pooled / jnp.maximum(cnt, 1.0)
        hh = jax.nn.silu(pooled @ wfc_ref[...] + bfc_ref[...])
        logits = hh @ wo_ref[...] + bo_ref[...]
        l4 = logits[:, 0:4]
        mx = jnp.max(l4, axis=1, keepdims=True)
        lse = jnp.log(jnp.sum(jnp.exp(l4 - mx), axis=1, keepdims=True))
        res = l4 - mx - lse
        pad = jnp.zeros((ng, 124), jnp.float32)
        out_ref[...] = jnp.concatenate([res, pad], axis=1)

    return pl.pallas_call(
        body,
        out_shape=jax.ShapeDtypeStruct((ng, 128), jnp.float32),
        interpret=_INTERPRET,
    )(h, batch8, wfc, bfc, wout_pad, bout_pad)


def _gather_rows(table, idx, width):
    n_rows = idx.shape[0]
    per_tile = n_rows // NW
    n_chunks = per_tile // GCHUNK
    mesh = plsc.VectorSubcoreMesh(core_axis_name="c", subcore_axis_name="s")

    nb = 6
    assert n_chunks % nb == 0

    @functools.partial(
        pl.kernel,
        mesh=mesh,
        out_type=jax.ShapeDtypeStruct((n_rows, width), jnp.float32),
        scratch_types=[
            pltpu.VMEM((per_tile,), jnp.int32),
        ] + [pltpu.VMEM((GCHUNK, width), jnp.float32) for _ in range(nb)]
        + [pltpu.SemaphoreType.DMA for _ in range(2 * nb)],
    )
    def k(table_hbm, idx_hbm, out_hbm, idx_all, *rest):
        bufs = rest[:nb]
        sg = rest[nb:2 * nb]
        sw = rest[2 * nb:3 * nb]
        cid = lax.axis_index("c")
        sid = lax.axis_index("s")
        base = pl.multiple_of((sid * NC + cid) * per_tile, GCHUNK)
        pltpu.sync_copy(idx_hbm.at[pl.ds(base, per_tile)], idx_all)

        def gsrc(j):
            o = pl.multiple_of(j * GCHUNK, GCHUNK)
            return table_hbm.at[idx_all.at[pl.ds(o, GCHUNK)]]

        def wdst(j):
            r = pl.multiple_of(base + j * GCHUNK, GCHUNK)
            return out_hbm.at[pl.ds(r, GCHUNK)]

        for b in range(nb):
            pltpu.async_copy(gsrc(b), bufs[b], sg[b])

        def body(jj, carry):
            j0 = jj * nb
            for b in range(nb):
                pltpu.make_async_copy(gsrc(j0 + b), bufs[b], sg[b]).wait()
                pltpu.async_copy(bufs[b], wdst(j0 + b), sw[b])
            for b in range(nb):
                pltpu.make_async_copy(bufs[b], wdst(j0 + b), sw[b]).wait()

                @pl.when(jj < n_chunks // nb - 1)
                def _(b=b):
                    pltpu.async_copy(gsrc(j0 + nb + b), bufs[b], sg[b])

            return carry

        lax.fori_loop(0, n_chunks // nb, body, 0)

    return k(table, idx)


def _scatter_add(msgs, dst, n_nodes, core_of):
    epad = dst.shape[0]
    per_tile = epad // NW
    n_chunks = per_tile // GCHUNK
    stripe = (n_nodes // NS) // 8 * 8
    last_stripe = n_nodes - (NS - 1) * stripe
    na = len(msgs)
    widths = [m.shape[1] for m in msgs]
    w = widths[0]
    assert all(wi == w for wi in widths)
    groups = [[a for a in range(na) if core_of[a] == c] for c in range(NC)]
    assert max(len(g) for g in groups) == 1
    mesh = plsc.VectorSubcoreMesh(core_axis_name="c", subcore_axis_name="s")

    zeros = jnp.zeros((last_stripe, w), jnp.float32)
    dst3 = dst.reshape(NW, n_chunks, GCHUNK)

    nb = 2
    assert n_chunks % nb == 0
    scratch = (
        [pltpu.VMEM((n_chunks, GCHUNK), jnp.int32)]
        + [pltpu.VMEM((GCHUNK, w), jnp.float32) for _ in range(nb)]
        + [pltpu.VMEM_SHARED((n_nodes, w), jnp.float32)]
        + [pltpu.SemaphoreType.DMA for _ in range(nb)]
    )

    @functools.partial(
        pl.kernel,
        mesh=mesh,
        out_type=[jax.ShapeDtypeStruct((n_nodes, w), jnp.float32)
                  for _ in range(na)],
        scratch_types=scratch,
    )
    def k(*refs):
        msg_refs = refs[0:na]
        dst_ref = refs[na]
        zero_ref = refs[na + 1]
        out_refs = refs[na + 2:2 * na + 2]
        rest = refs[2 * na + 2:]
        idx2d = rest[0]
        bufs = rest[1:1 + nb]
        acc = rest[1 + nb]
        sl = rest[2 + nb:2 + 2 * nb]

        cid = lax.axis_index("c")
        sid = lax.axis_index("s")
        wid = sid * NC + cid
        base = pl.multiple_of(wid * per_tile, GCHUNK)
        off = pl.multiple_of(sid * stripe, 8)

        for length, pred in ((stripe, sid < NS - 1),
                             (last_stripe, sid == NS - 1)):

            @pl.when(pred)
            def _(length=length):
                pltpu.sync_copy(zero_ref.at[pl.ds(0, length)],
                                acc.at[pl.ds(off, length)])

        plsc.subcore_barrier()

        for c in range(NC):
            if not groups[c]:
                continue
            a = groups[c][0]

            @pl.when(cid == c)
            def _(a=a):
                mref = msg_refs[a]
                pltpu.sync_copy(dst_ref.at[wid], idx2d)

                def lsrc(j):
                    r = pl.multiple_of(base + j * GCHUNK, GCHUNK)
                    return mref.at[pl.ds(r, GCHUNK)]

                for b in range(nb):
                    pltpu.async_copy(lsrc(b), bufs[b], sl[b])

                def body(jj, carry):
                    j0 = jj * nb
                    for b in range(nb):
                        pltpu.make_async_copy(
                            lsrc(j0 + b), bufs[b], sl[b]).wait()
                        pltpu.sync_copy(bufs[b], acc.at[idx2d.at[j0 + b]],
                                        add=True)

                        @pl.when(jj < n_chunks // nb - 1)
                        def _(b=b):
                            pltpu.async_copy(lsrc(j0 + nb + b), bufs[b], sl[b])

                    return carry

                lax.fori_loop(0, n_chunks // nb, body, 0)

        plsc.subcore_barrier()

        for c in range(NC):
            if not groups[c]:
                continue
            a = groups[c][0]
            for length, pred in ((stripe, sid < NS - 1),
                                 (last_stripe, sid == NS - 1)):

                @pl.when(jnp.logical_and(cid == c, pred))
                def _(a=a, length=length):
                    pltpu.sync_copy(acc.at[pl.ds(off, length)],
                                    out_refs[a].at[pl.ds(off, length)])

    return k(*msgs, dst3, zeros)


def kernel(x, edge_attr, inv_edge_attr, params, wyckoff, edge_index,
           inv_edge_index, inv_edge_type, batch):
    n = x.shape[0]
    e_r = edge_attr.shape[0]
    e_i = inv_edge_attr.shape[0]
    ne = e_r + e_i
    epad = _cdiv(ne, NW * GCHUNK * 2) * NW * GCHUNK * 2
    ng = 64
    f32 = jnp.float32

    ea8 = jnp.concatenate([
        jnp.concatenate([edge_attr, jnp.zeros((e_r, 5), f32)], axis=1),
        jnp.concatenate([inv_edge_attr, jnp.ones((e_i, 1), f32),
                         inv_edge_type.astype(f32)[:, None],
                         jnp.zeros((e_i, 3), f32)], axis=1),
        jnp.concatenate([jnp.ones((epad - ne, 1), f32),
                         jnp.zeros((epad - ne, 7), f32)], axis=1),
    ], axis=0)

    zpad = jnp.zeros((epad - ne,), jnp.int32)
    src = jnp.concatenate([edge_index[0], inv_edge_index[0], zpad])
    dst = jnp.concatenate([edge_index[1], inv_edge_index[1], zpad])
    idx_ds = jnp.concatenate([dst, src])

    x_pad = jnp.concatenate([x, jnp.zeros((n, 128 - x.shape[1]), f32)], axis=1)
    wyk8 = jnp.broadcast_to(wyckoff.astype(jnp.int32)[:, None], (n, 8))
    batch8 = jnp.broadcast_to(batch.astype(jnp.int32)[:, None], (n, 8))

    wa1 = jnp.concatenate([params["W_atom"][:x.shape[1]],
                           jnp.zeros((128 - x.shape[1], C), f32)], axis=0)
    t2 = params["wyckoff_table"] @ params["W_atom"][x.shape[1]:]
    t2 = jnp.concatenate([t2, jnp.zeros((128 - t2.shape[0], C), f32)], axis=0)

    w2 = params["W_rbf"][C:]
    u_vec = (params["W_inv"] @ w2).reshape(1, C)
    w_vec = (params["b_inv"] @ w2).reshape(1, C)

    equi = params["equi"]
    wtp = equi["Wtp"]
    wtp_g = [
        jnp.concatenate([wtp[j::9] for j in range(4)], axis=0),
        jnp.concatenate([wtp[j::9] for j in range(4, 8)], axis=0),
        jnp.concatenate([wtp[8::9], jnp.zeros((96, C), f32)], axis=0),
    ]
    wn_pad = jnp.concatenate([equi["Wn"], jnp.zeros((C, 96), f32)], axis=1)

    wout_pad = jnp.concatenate([params["W_out"],
                                jnp.zeros((C, 124), f32)], axis=1)
    bout_pad = jnp.concatenate([params["b_out"],
                                jnp.zeros((124,), f32)]).reshape(1, 128)

    h = _embed_call(x_pad, wyk8, wa1, t2, params["b_atom"].reshape(1, C))
    ef = _ef_call(ea8, params["W_rbf"][:C], u_vec, w_vec,
                  params["b_rbf"].reshape(1, C), ne)

    def conv(h, p):
        hds = _gather_rows(_pack_call(h), idx_ds, 128)
        msg0, msg1 = _conv_edge_call(hds, ef, p, ne)
        agg0, agg1 = _scatter_add([msg0, msg1], dst, n, [0, 1])
        return _conv_node_call(agg0, agg1, h, p)

    h = conv(h, params["conv0"])

    hn = _hn_call(h, wn_pad)
    hns = _gather_rows(hn, src, 128)
    tps = _equi_edge_call(hns, ef, ea8, equi["Wes"],
                          equi["bes"].reshape(1, 32), ne)
    aggs01 = _scatter_add(tps[:2], dst, n, [0, 1])
    aggs2 = _scatter_add(tps[2:], dst, n, [0])
    h = _equi_node_call(h, aggs01 + aggs2, wtp_g, equi["gln"], equi["bln"])

    h = conv(h, params["conv1"])
    h = conv(h, params["conv2"])

    out = _pool_call(h, batch8, params["W_fc"], params["b_fc"].reshape(1, C),
                     wout_pad, bout_pad, ng)
    return out[:, :4]


# speedup vs baseline: 1.9714x; 1.0226x over previous
import functools

import jax
import jax.numpy as jnp
import numpy as np
from jax import lax
from jax.experimental import pallas as pl
from jax.experimental.pallas import tpu as pltpu
from jax.experimental.pallas import tpu_sc as plsc

C = 256
EBLK = 512
NC, NS = 2, 16
NW = NC * NS
GCHUNK = 128

_INTERPRET = False


def _cdiv(a, b):
    return (a + b - 1) // b


def _embed_call(x_pad, wyk8, wa1, t2, b_atom):
    n = x_pad.shape[0]

    def body(x_ref, wy_ref, wa_ref, t2_ref, b_ref, out_ref):
        x = x_ref[...]
        ids = wy_ref[...][:, 0:1]
        oh = (ids == lax.broadcasted_iota(jnp.int32, (1, 128), 1)).astype(jnp.float32)
        out_ref[...] = x @ wa_ref[...] + oh @ t2_ref[...] + b_ref[...]

    return pl.pallas_call(
        body,
        out_shape=jax.ShapeDtypeStruct((n, C), jnp.float32),
        interpret=_INTERPRET,
    )(x_pad, wyk8, wa1, t2, b_atom)


def _ef_call(ea8, wr1, u_vec, w_vec, b_rbf, n_real):
    epad = ea8.shape[0]
    grid = epad // EBLK
    gamma2 = float((255.0 / 4.0) ** 2)

    def body(ea_ref, wr_ref, u_ref, w_ref, b_ref, out_ref):
        ea = ea_ref[...]
        vx, vy, vz = ea[:, 0:1], ea[:, 1:2], ea[:, 2:3]
        nrm = jnp.sqrt(vx * vx + vy * vy + vz * vz)
        is_inv = ea[:, 3:4]
        ity = ea[:, 4:5]
        s = jnp.where(is_inv > 0.5, ity, -0.75 / nrm)
        cent = -4.0 + lax.broadcasted_iota(
            jnp.int32, (1, C), 1).astype(jnp.float32) * (4.0 / 255.0)
        r = jnp.exp(-gamma2 * (s - cent) ** 2)
        pre = r @ wr_ref[...] + is_inv * (ity * u_ref[...] + w_ref[...]) + b_ref[...]
        out_ref[...] = jax.nn.softplus(pre)

    return pl.pallas_call(
        body,
        grid=(grid,),
        in_specs=[
            pl.BlockSpec((EBLK, 8), lambda i: (i, 0)),
            pl.BlockSpec((C, C), lambda i: (0, 0)),
            pl.BlockSpec((1, C), lambda i: (0, 0)),
            pl.BlockSpec((1, C), lambda i: (0, 0)),
            pl.BlockSpec((1, C), lambda i: (0, 0)),
        ],
        out_specs=pl.BlockSpec((EBLK, C), lambda i: (i, 0)),
        out_shape=jax.ShapeDtypeStruct((epad, C), jnp.float32),
        interpret=_INTERPRET,
    )(ea8, wr1, u_vec, w_vec, b_rbf)


def _pack_call(h):
    n = h.shape[0]

    def body(h_ref, out_ref):
        hv = h_ref[...]
        a = lax.bitcast_convert_type(
            hv[:, :128].astype(jnp.bfloat16), jnp.uint16).astype(jnp.uint32)
        b = lax.bitcast_convert_type(
            hv[:, 128:].astype(jnp.bfloat16), jnp.uint16).astype(jnp.uint32)
        out_ref[...] = lax.bitcast_convert_type(a | (b << 16), jnp.float32)

    return pl.pallas_call(
        body,
        out_shape=jax.ShapeDtypeStruct((n, 128), jnp.float32),
        interpret=_INTERPRET,
    )(h)


def _conv_edge_call(hds, ef, p, n_real, blk0):
    epad = hds.shape[0] // 2
    grid = epad // EBLK
    nb = grid
    inv_s = float(1.0 / np.sqrt(3.0 * C))

    def body(hd_ref, hs_ref, ef_ref, wq_ref, wk_ref, wv_ref, we_ref,
             bq_ref, bk_ref, bv_ref, be_ref, wm1_ref, wm2_ref, wm3_ref,
             bm_ref, g1_ref, b1_ref, wmsg_ref, bmsg_ref, g2_ref, b2_ref,
             m0_ref, m1_ref):
        bf = jnp.bfloat16
        f32 = jnp.float32

        def dot16(a, b_ref):
            return jnp.dot(a.astype(bf), b_ref[...],
                           preferred_element_type=f32)

        def unpack(ref):
            u = lax.bitcast_convert_type(ref[...], jnp.uint32)
            lo = lax.bitcast_convert_type(
                (u & 0xFFFF).astype(jnp.uint16), bf)
            hi = lax.bitcast_convert_type(
                (u >> 16).astype(jnp.uint16), bf)
            return lo, hi

        def dot2(lohi, w_ref):
            w = w_ref[...]
            return (jnp.dot(lohi[0], w[:128], preferred_element_type=f32)
                    + jnp.dot(lohi[1], w[128:], preferred_element_type=f32))

        hd = unpack(hd_ref)
        hs = unpack(hs_ref)
        ef_b = ef_ref[...]
        q_d = dot2(hd, wq_ref) + bq_ref[...]
        k_d = dot2(hd, wk_ref) + bk_ref[...]
        k_s = dot2(hs, wk_ref) + bk_ref[...]
        v_d = dot2(hd, wv_ref) + bv_ref[...]
        v_s = dot2(hs, wv_ref) + bv_ref[...]
        e = dot16(ef_b, we_ref) + be_ref[...]
        alpha = jnp.concatenate([q_d * k_d, q_d * k_s, q_d * e], axis=1) * inv_s
        mu = jnp.mean(alpha, axis=1, keepdims=True)
        var = jnp.mean(alpha * alpha, axis=1, keepdims=True) - mu * mu
        gate = jax.nn.sigmoid((alpha - mu) * lax.rsqrt(var + 1e-5)
                              * g1_ref[...] + b1_ref[...])
        m = (dot16(v_d, wm1_ref) + dot16(v_s, wm2_ref) + dot16(e, wm3_ref)
             + bm_ref[...])
        m = m * gate
        msg = dot16(m, wmsg_ref) + bmsg_ref[...]
        mu2 = jnp.mean(msg, axis=1, keepdims=True)
        var2 = jnp.mean(msg * msg, axis=1, keepdims=True) - mu2 * mu2
        msg = (msg - mu2) * lax.rsqrt(var2 + 1e-5) * g2_ref[...] + b2_ref[...]
        eid = ((pl.program_id(0) + blk0) * EBLK
               + lax.broadcasted_iota(jnp.int32, (EBLK, 1), 0))
        msg = jnp.where(eid < n_real, msg, 0.0)
        m0_ref[...] = msg[:, :128]
        m1_ref[...] = msg[:, 128:]

    w_spec = lambda shape: pl.BlockSpec(shape, lambda i: (0, 0))
    out = pl.pallas_call(
        body,
        grid=(grid,),
        in_specs=[
            pl.BlockSpec((EBLK, 128), lambda i: (i, 0)),
            pl.BlockSpec((EBLK, 128), lambda i: (i + nb, 0)),
            pl.BlockSpec((EBLK, C), lambda i: (i + blk0, 0)),
            w_spec((C, C)), w_spec((C, C)), w_spec((C, C)), w_spec((C, C)),
            w_spec((1, C)), w_spec((1, C)), w_spec((1, C)), w_spec((1, C)),
            w_spec((C, 3 * C)), w_spec((C, 3 * C)), w_spec((C, 3 * C)),
            w_spec((1, 3 * C)), w_spec((1, 3 * C)), w_spec((1, 3 * C)),
            w_spec((3 * C, C)), w_spec((1, C)), w_spec((1, C)), w_spec((1, C)),
        ],
        out_specs=[
            pl.BlockSpec((EBLK, 128), lambda i: (i, 0)),
            pl.BlockSpec((EBLK, 128), lambda i: (i, 0)),
        ],
        out_shape=[
            jax.ShapeDtypeStruct((epad, 128), jnp.float32),
            jax.ShapeDtypeStruct((epad, 128), jnp.float32),
        ],
        interpret=_INTERPRET,
    )
    bf = jnp.bfloat16
    out = out(hds, hds, ef,
              p["Wq"].astype(bf), p["Wk"].astype(bf), p["Wv"].astype(bf),
              p["We"].astype(bf),
              p["bq"].reshape(1, C), p["bk"].reshape(1, C),
              p["bv"].reshape(1, C), p["be"].reshape(1, C),
              p["Wm"][:C].astype(bf), p["Wm"][C:2 * C].astype(bf),
              p["Wm"][2 * C:].astype(bf),
              p["bm"].reshape(1, 3 * C), p["g1"].reshape(1, 3 * C),
              p["b1"].reshape(1, 3 * C),
              p["Wmsg"].astype(bf), p["bmsg"].reshape(1, C),
              p["g2"].reshape(1, C), p["b2"].reshape(1, C))
    return out


def _conv_node_call(agg0, agg1, h, p):
    n = h.shape[0]

    def body(a0_ref, a1_ref, h_ref, wc0_ref, wc1_ref, bc_ref, g_ref, b_ref,
             out_ref):
        out = a0_ref[...] @ wc0_ref[...] + a1_ref[...] @ wc1_ref[...] + bc_ref[...]
        mu = jnp.mean(out, axis=0, keepdims=True)
        var = jnp.mean(out * out, axis=0, keepdims=True) - mu * mu
        bn = (out - mu) * lax.rsqrt(var + 1e-5) * g_ref[...] + b_ref[...]
        out_ref[...] = jax.nn.softplus(h_ref[...] + bn)

    return pl.pallas_call(
        body,
        grid=(2,),
        in_specs=[
            pl.BlockSpec((n, 128), lambda j: (0, 0)),
            pl.BlockSpec((n, 128), lambda j: (0, 0)),
            pl.BlockSpec((n, 128), lambda j: (0, j)),
            pl.BlockSpec((128, 128), lambda j: (0, j)),
            pl.BlockSpec((128, 128), lambda j: (0, j)),
            pl.BlockSpec((1, 128), lambda j: (0, j)),
            pl.BlockSpec((1, 128), lambda j: (0, j)),
            pl.BlockSpec((1, 128), lambda j: (0, j)),
        ],
        out_specs=pl.BlockSpec((n, 128), lambda j: (0, j)),
        out_shape=jax.ShapeDtypeStruct((n, C), jnp.float32),
        interpret=_INTERPRET,
    )(agg0, agg1, h, p["Wc"][:128], p["Wc"][128:], p["bc"].reshape(1, C),
      p["gbn"].reshape(1, C), p["bbn"].reshape(1, C))


def _hn_call(h, wn_pad):
    n = h.shape[0]

    def body(h_ref, w_ref, out_ref):
        out_ref[...] = h_ref[...] @ w_ref[...]

    return pl.pallas_call(
        body,
        out_shape=jax.ShapeDtypeStruct((n, 128), jnp.float32),
        interpret=_INTERPRET,
    )(h, wn_pad)


def _equi_edge_call(hns, ef, ea8, wes, bes, n_real):
    epad = ef.shape[0]
    grid = epad // EBLK
    c0, c1, c2, c3, c4 = (0.28209479177, 0.48860251190, 1.09254843059,
                          0.31539156525, 0.54627421529)

    def body(hn_ref, ef_ref, ea_ref, wes_ref, bes_ref, *out_refs):
        ea = ea_ref[...]
        vx, vy, vz = ea[:, 0:1], ea[:, 1:2], ea[:, 2:3]
        nrm = jnp.sqrt(vx * vx + vy * vy + vz * vz) + 1e-8
        x = vx / nrm
        y = vy / nrm
        z = vz / nrm
        es = jax.nn.silu(ef_ref[...] @ wes_ref[...] + bes_ref[...])
        m = hn_ref[...][:, :32] * es
        eid = pl.program_id(0) * EBLK + lax.broadcasted_iota(jnp.int32, (EBLK, 1), 0)
        m = jnp.where(eid < n_real, m, 0.0)
        sh = [jnp.full_like(x, c0), c1 * y, c1 * z, c1 * x,
              c2 * x * y, c2 * y * z, c3 * (3.0 * z * z - 1.0),
              c2 * x * z, c4 * (x * x - y * y)]
        out_refs[0][...] = jnp.concatenate([m * sh[j] for j in range(4)], axis=1)
        out_refs[1][...] = jnp.concatenate([m * sh[j] for j in range(4, 8)], axis=1)
        out_refs[2][...] = jnp.concatenate(
            [m * sh[8], jnp.zeros((EBLK, 96), jnp.float32)], axis=1)

    return pl.pallas_call(
        body,
        grid=(grid,),
        in_specs=[
            pl.BlockSpec((EBLK, 128), lambda i: (i, 0)),
            pl.BlockSpec((EBLK, C), lambda i: (i, 0)),
            pl.BlockSpec((EBLK, 8), lambda i: (i, 0)),
            pl.BlockSpec((C, 32), lambda i: (0, 0)),
            pl.BlockSpec((1, 32), lambda i: (0, 0)),
        ],
        out_specs=[pl.BlockSpec((EBLK, 128), lambda i: (i, 0))] * 3,
        out_shape=[jax.ShapeDtypeStruct((epad, 128), jnp.float32)] * 3,
        interpret=_INTERPRET,
    )(hns, ef, ea8, wes, bes)


def _equi_node_call(h, aggs, wtps, gln, bln):
    n = h.shape[0]

    def body(*refs):
        h_ref = refs[0]
        agg_refs = refs[1:4]
        wtp_refs = refs[4:7]
        g_ref, b_ref = refs[7], refs[8]
        out_ref = refs[9]
        acc = agg_refs[0][...] @ wtp_refs[0][...]
        for j in range(1, 3):
            acc = acc + agg_refs[j][...] @ wtp_refs[j][...]
        mu = jnp.mean(acc, axis=1, keepdims=True)
        var = jnp.mean(acc * acc, axis=1, keepdims=True) - mu * mu
        ln = (acc - mu) * lax.rsqrt(var + 1e-5) * g_ref[...] + b_ref[...]
        out_ref[...] = h_ref[...] + jax.nn.silu(ln)

    rblk = 1000 if n % 1000 == 0 else n
    return pl.pallas_call(
        body,
        grid=(n // rblk,),
        in_specs=[pl.BlockSpec((rblk, C), lambda i: (i, 0))]
        + [pl.BlockSpec((rblk, 128), lambda i: (i, 0))] * 3
        + [pl.BlockSpec((128, C), lambda i: (0, 0))] * 3
        + [pl.BlockSpec((1, C), lambda i: (0, 0))] * 2,
        out_specs=pl.BlockSpec((rblk, C), lambda i: (i, 0)),
        out_shape=jax.ShapeDtypeStruct((n, C), jnp.float32),
        interpret=_INTERPRET,
    )(h, *aggs, *wtps, gln.reshape(1, C), bln.reshape(1, C))


def _pool_call(h, batch8, wfc, bfc, wout_pad, bout_pad, ng):
    n = h.shape[0]

    def body(h_ref, b_ref, wfc_ref, bfc_ref, wo_ref, bo_ref, out_ref):
        ids = b_ref[...][:, 0:1]
        oh = (ids == lax.broadcasted_iota(jnp.int32, (1, ng), 1)).astype(jnp.float32)
        pooled = lax.dot_general(oh, h_ref[...], (((0,), (0,)), ((), ())))
        ones = jnp.ones((n, 1), jnp.float32)
        cnt = lax.dot_general(oh, ones, (((0,), (0,)), ((), ())))
        pooled = pooled / jnp.maximum(cnt, 1.0)
        hh = jax.nn.silu(pooled @ wfc_ref[...] + bfc_ref[...])
        logits = hh @ wo_ref[...] + bo_ref[...]
        l4 = logits[:, 0:4]
        mx = jnp.max(l4, axis=1, keepdims=True)
        lse = jnp.log(jnp.sum(jnp.exp(l4 - mx), axis=1, keepdims=True))
        res = l4 - mx - lse
        pad = jnp.zeros((ng, 124), jnp.float32)
        out_ref[...] = jnp.concatenate([res, pad], axis=1)

    return pl.pallas_call(
        body,
        out_shape=jax.ShapeDtypeStruct((ng, 128), jnp.float32),
        interpret=_INTERPRET,
    )(h, batch8, wfc, bfc, wout_pad, bout_pad)


def _gather_rows(table, idx, width):
    n_rows = idx.shape[0]
    per_tile = n_rows // NW
    n_chunks = per_tile // GCHUNK
    mesh = plsc.VectorSubcoreMesh(core_axis_name="c", subcore_axis_name="s")

    nb = 6 if width <= 128 else 3
    assert n_chunks % nb == 0

    @functools.partial(
        pl.kernel,
        mesh=mesh,
        out_type=jax.ShapeDtypeStruct((n_rows, width), jnp.float32),
        scratch_types=[
            pltpu.VMEM((per_tile,), jnp.int32),
        ] + [pltpu.VMEM((GCHUNK, width), jnp.float32) for _ in range(nb)]
        + [pltpu.SemaphoreType.DMA for _ in range(2 * nb)],
    )
    def k(table_hbm, idx_hbm, out_hbm, idx_all, *rest):
        bufs = rest[:nb]
        sg = rest[nb:2 * nb]
        sw = rest[2 * nb:3 * nb]
        cid = lax.axis_index("c")
        sid = lax.axis_index("s")
        base = pl.multiple_of((sid * NC + cid) * per_tile, GCHUNK)
        pltpu.sync_copy(idx_hbm.at[pl.ds(base, per_tile)], idx_all)

        def gsrc(j):
            o = pl.multiple_of(j * GCHUNK, GCHUNK)
            return table_hbm.at[idx_all.at[pl.ds(o, GCHUNK)]]

        def wdst(j):
            r = pl.multiple_of(base + j * GCHUNK, GCHUNK)
            return out_hbm.at[pl.ds(r, GCHUNK)]

        for b in range(nb):
            pltpu.async_copy(gsrc(b), bufs[b], sg[b])

        def body(jj, carry):
            j0 = jj * nb
            for b in range(nb):
                pltpu.make_async_copy(gsrc(j0 + b), bufs[b], sg[b]).wait()
                pltpu.async_copy(bufs[b], wdst(j0 + b), sw[b])
            for b in range(nb):
                pltpu.make_async_copy(bufs[b], wdst(j0 + b), sw[b]).wait()

                @pl.when(jj < n_chunks // nb - 1)
                def _(b=b):
                    pltpu.async_copy(gsrc(j0 + nb + b), bufs[b], sg[b])

            return carry

        lax.fori_loop(0, n_chunks // nb, body, 0)

    return k(table, idx)


def _scatter_add(msgs, dst, n_nodes, core_of):
    msgs = [m if isinstance(m, tuple) else (m,) for m in msgs]
    nh = len(msgs[0])
    assert all(len(m) == nh for m in msgs) and NW % nh == 0
    tiles_per_half = NW // nh
    epad = dst.shape[0]
    per_tile = epad // NW
    n_chunks = per_tile // GCHUNK
    stripe = (n_nodes // NS) // 8 * 8
    last_stripe = n_nodes - (NS - 1) * stripe
    na = len(msgs)
    widths = [m[0].shape[1] for m in msgs]
    w = widths[0]
    assert all(wi == w for wi in widths)
    groups = [[a for a in range(na) if core_of[a] == c] for c in range(NC)]
    assert max(len(g) for g in groups) == 1
    mesh = plsc.VectorSubcoreMesh(core_axis_name="c", subcore_axis_name="s")

    zeros = jnp.zeros((last_stripe, w), jnp.float32)
    dst3 = dst.reshape(NW, n_chunks, GCHUNK)

    nb = 2
    assert n_chunks % nb == 0
    scratch = (
        [pltpu.VMEM((n_chunks, GCHUNK), jnp.int32)]
        + [pltpu.VMEM((GCHUNK, w), jnp.float32) for _ in range(nb)]
        + [pltpu.VMEM_SHARED((n_nodes, w), jnp.float32)]
        + [pltpu.SemaphoreType.DMA for _ in range(nb)]
    )

    @functools.partial(
        pl.kernel,
        mesh=mesh,
        out_type=[jax.ShapeDtypeStruct((n_nodes, w), jnp.float32)
                  for _ in range(na)],
        scratch_types=scratch,
    )
    def k(*refs):
        nm = na * nh
        msg_refs = refs[0:nm]
        dst_ref = refs[nm]
        zero_ref = refs[nm + 1]
        out_refs = refs[nm + 2:nm + na + 2]
        rest = refs[nm + na + 2:]
        idx2d = rest[0]
        bufs = rest[1:1 + nb]
        acc = rest[1 + nb]
        sl = rest[2 + nb:2 + 2 * nb]

        cid = lax.axis_index("c")
        sid = lax.axis_index("s")
        wid = sid * NC + cid
        base = pl.multiple_of(wid * per_tile, GCHUNK)
        off = pl.multiple_of(sid * stripe, 8)

        for length, pred in ((stripe, sid < NS - 1),
                             (last_stripe, sid == NS - 1)):

            @pl.when(pred)
            def _(length=length):
                pltpu.sync_copy(zero_ref.at[pl.ds(0, length)],
                                acc.at[pl.ds(off, length)])

        plsc.subcore_barrier()
        pltpu.sync_copy(dst_ref.at[wid], idx2d)

        def accum(mref, mbase):
            def lsrc(j):
                r = pl.multiple_of(mbase + j * GCHUNK, GCHUNK)
                return mref.at[pl.ds(r, GCHUNK)]

            for b in range(nb):
                pltpu.async_copy(lsrc(b), bufs[b], sl[b])

            def body(jj, carry):
                j0 = jj * nb
                for b in range(nb):
                    pltpu.make_async_copy(lsrc(j0 + b), bufs[b], sl[b]).wait()
                    pltpu.sync_copy(bufs[b], acc.at[idx2d.at[j0 + b]],
                                    add=True)

                    @pl.when(jj < n_chunks // nb - 1)
                    def _(b=b):
                        pltpu.async_copy(lsrc(j0 + nb + b), bufs[b], sl[b])

                return carry

            lax.fori_loop(0, n_chunks // nb, body, 0)

        for c in range(NC):
            if not groups[c]:
                continue
            a = groups[c][0]
            for hh in range(nh):
                lo = hh * tiles_per_half
                pred = jnp.logical_and(
                    cid == c,
                    jnp.logical_and(wid >= lo, wid < lo + tiles_per_half))

                @pl.when(pred)
                def _(a=a, hh=hh, lo=lo):
                    mbase = pl.multiple_of((wid - lo) * per_tile, GCHUNK)
                    accum(msg_refs[a * nh + hh], mbase)

        plsc.subcore_barrier()

        for c in range(NC):
            if not groups[c]:
                continue
            a = groups[c][0]
            for length, pred in ((stripe, sid < NS - 1),
                                 (last_stripe, sid == NS - 1)):

                @pl.when(jnp.logical_and(cid == c, pred))
                def _(a=a, length=length):
                    pltpu.sync_copy(acc.at[pl.ds(off, length)],
                                    out_refs[a].at[pl.ds(off, length)])

    return k(*[m for t in msgs for m in t], dst3, zeros)


def kernel(x, edge_attr, inv_edge_attr, params, wyckoff, edge_index,
           inv_edge_index, inv_edge_type, batch):
    n = x.shape[0]
    e_r = edge_attr.shape[0]
    e_i = inv_edge_attr.shape[0]
    ne = e_r + e_i
    epad = _cdiv(ne, NW * GCHUNK * 2) * NW * GCHUNK * 2
    ng = 64
    f32 = jnp.float32

    ea8 = jnp.concatenate([
        jnp.concatenate([edge_attr, jnp.zeros((e_r, 5), f32)], axis=1),
        jnp.concatenate([inv_edge_attr, jnp.ones((e_i, 1), f32),
                         inv_edge_type.astype(f32)[:, None],
                         jnp.zeros((e_i, 3), f32)], axis=1),
        jnp.concatenate([jnp.ones((epad - ne, 1), f32),
                         jnp.zeros((epad - ne, 7), f32)], axis=1),
    ], axis=0)

    zpad = jnp.zeros((epad - ne,), jnp.int32)
    src = jnp.concatenate([edge_index[0], inv_edge_index[0], zpad])
    dst = jnp.concatenate([edge_index[1], inv_edge_index[1], zpad])
    ehalf = epad // 2
    idx_a = jnp.concatenate([dst[:ehalf], src[:ehalf]])
    idx_b = jnp.concatenate([dst[ehalf:], src[ehalf:]])

    x_pad = jnp.concatenate([x, jnp.zeros((n, 128 - x.shape[1]), f32)], axis=1)
    wyk8 = jnp.broadcast_to(wyckoff.astype(jnp.int32)[:, None], (n, 8))
    batch8 = jnp.broadcast_to(batch.astype(jnp.int32)[:, None], (n, 8))

    wa1 = jnp.concatenate([params["W_atom"][:x.shape[1]],
                           jnp.zeros((128 - x.shape[1], C), f32)], axis=0)
    t2 = params["wyckoff_table"] @ params["W_atom"][x.shape[1]:]
    t2 = jnp.concatenate([t2, jnp.zeros((128 - t2.shape[0], C), f32)], axis=0)

    w2 = params["W_rbf"][C:]
    u_vec = (params["W_inv"] @ w2).reshape(1, C)
    w_vec = (params["b_inv"] @ w2).reshape(1, C)

    equi = params["equi"]
    wtp = equi["Wtp"]
    wtp_g = [
        jnp.concatenate([wtp[j::9] for j in range(4)], axis=0),
        jnp.concatenate([wtp[j::9] for j in range(4, 8)], axis=0),
        jnp.concatenate([wtp[8::9], jnp.zeros((96, C), f32)], axis=0),
    ]
    wn_pad = jnp.concatenate([equi["Wn"], jnp.zeros((C, 96), f32)], axis=1)

    wout_pad = jnp.concatenate([params["W_out"],
                                jnp.zeros((C, 124), f32)], axis=1)
    bout_pad = jnp.concatenate([params["b_out"],
                                jnp.zeros((124,), f32)]).reshape(1, 128)

    h = _embed_call(x_pad, wyk8, wa1, t2, params["b_atom"].reshape(1, C))
    ef = _ef_call(ea8, params["W_rbf"][:C], u_vec, w_vec,
                  params["b_rbf"].reshape(1, C), ne)

    def conv(h, p):
        hp = _pack_call(h)
        hds_a = _gather_rows(hp, idx_a, 128)
        hds_b = _gather_rows(hp, idx_b, 128)
        ma0, ma1 = _conv_edge_call(hds_a, ef, p, ne, 0)
        mb0, mb1 = _conv_edge_call(hds_b, ef, p, ne, ehalf // EBLK)
        agg0, agg1 = _scatter_add([(ma0, mb0), (ma1, mb1)], dst, n, [0, 1])
        return _conv_node_call(agg0, agg1, h, p)

    h = conv(h, params["conv0"])

    hn = _hn_call(h, wn_pad)
    hns = _gather_rows(hn, src, 128)
    tps = _equi_edge_call(hns, ef, ea8, equi["Wes"],
                          equi["bes"].reshape(1, 32), ne)
    aggs01 = _scatter_add(tps[:2], dst, n, [0, 1])
    aggs2 = _scatter_add(tps[2:], dst, n, [0])
    h = _equi_node_call(h, aggs01 + aggs2, wtp_g, equi["gln"], equi["bln"])

    h = conv(h, params["conv1"])
    h = conv(h, params["conv2"])

    out = _pool_call(h, batch8, params["W_fc"], params["b_fc"].reshape(1, C),
                     wout_pad, bout_pad, ng)
    return out[:, :4]


# speedup vs baseline: 1.9737x; 1.0012x over previous
import functools

import jax
import jax.numpy as jnp
import numpy as np
from jax import lax
from jax.experimental import pallas as pl
from jax.experimental.pallas import tpu as pltpu
from jax.experimental.pallas import tpu_sc as plsc

C = 256
EBLK = 512
NC, NS = 2, 16
NW = NC * NS
GCHUNK = 128

def _cdiv(a, b):
    return (a + b - 1) // b


def _embed_call(x_pad, wyk8, wa1, t2, b_atom):
    n = x_pad.shape[0]

    def body(x_ref, wy_ref, wa_ref, t2_ref, b_ref, out_ref):
        x = x_ref[...]
        ids = wy_ref[...][:, 0:1]
        oh = (ids == lax.broadcasted_iota(jnp.int32, (1, 128), 1)).astype(jnp.float32)
        out_ref[...] = x @ wa_ref[...] + oh @ t2_ref[...] + b_ref[...]

    return pl.pallas_call(
        body,
        out_shape=jax.ShapeDtypeStruct((n, C), jnp.float32),
    )(x_pad, wyk8, wa1, t2, b_atom)


def _ef_call(ea8, wr1, u_vec, w_vec, b_rbf, n_real):
    epad = ea8.shape[0]
    grid = epad // EBLK
    gamma2 = float((255.0 / 4.0) ** 2)

    def body(ea_ref, wr_ref, u_ref, w_ref, b_ref, out_ref):
        ea = ea_ref[...]
        vx, vy, vz = ea[:, 0:1], ea[:, 1:2], ea[:, 2:3]
        nrm = jnp.sqrt(vx * vx + vy * vy + vz * vz)
        is_inv = ea[:, 3:4]
        ity = ea[:, 4:5]
        s = jnp.where(is_inv > 0.5, ity, -0.75 / nrm)
        cent = -4.0 + lax.broadcasted_iota(
            jnp.int32, (1, C), 1).astype(jnp.float32) * (4.0 / 255.0)
        r = jnp.exp(-gamma2 * (s - cent) ** 2)
        pre = r @ wr_ref[...] + is_inv * (ity * u_ref[...] + w_ref[...]) + b_ref[...]
        out_ref[...] = jax.nn.softplus(pre)

    return pl.pallas_call(
        body,
        grid=(grid,),
        in_specs=[
            pl.BlockSpec((EBLK, 8), lambda i: (i, 0)),
            pl.BlockSpec((C, C), lambda i: (0, 0)),
            pl.BlockSpec((1, C), lambda i: (0, 0)),
            pl.BlockSpec((1, C), lambda i: (0, 0)),
            pl.BlockSpec((1, C), lambda i: (0, 0)),
        ],
        out_specs=pl.BlockSpec((EBLK, C), lambda i: (i, 0)),
        out_shape=jax.ShapeDtypeStruct((epad, C), jnp.float32),
    )(ea8, wr1, u_vec, w_vec, b_rbf)


def _pack_call(h):
    n = h.shape[0]

    def body(h_ref, out_ref):
        hv = h_ref[...]
        a = lax.bitcast_convert_type(
            hv[:, :128].astype(jnp.bfloat16), jnp.uint16).astype(jnp.uint32)
        b = lax.bitcast_convert_type(
            hv[:, 128:].astype(jnp.bfloat16), jnp.uint16).astype(jnp.uint32)
        out_ref[...] = lax.bitcast_convert_type(a | (b << 16), jnp.float32)

    return pl.pallas_call(
        body,
        out_shape=jax.ShapeDtypeStruct((n, 128), jnp.float32),
    )(h)


def _conv_edge_call(hds, ef, p, n_real, blk0):
    epad = hds.shape[0] // 2
    grid = epad // EBLK
    nb = grid
    inv_s = float(1.0 / np.sqrt(3.0 * C))

    def body(hd_ref, hs_ref, ef_ref, wq_ref, wk_ref, wv_ref, we_ref,
             bq_ref, bk_ref, bv_ref, be_ref, wm1_ref, wm2_ref, wm3_ref,
             bm_ref, g1_ref, b1_ref, wmsg_ref, bmsg_ref, g2_ref, b2_ref,
             m0_ref, m1_ref):
        bf = jnp.bfloat16
        f32 = jnp.float32

        def dot16(a, b_ref):
            return jnp.dot(a.astype(bf), b_ref[...],
                           preferred_element_type=f32)

        def unpack(ref):
            u = lax.bitcast_convert_type(ref[...], jnp.uint32)
            lo = lax.bitcast_convert_type(
                (u & 0xFFFF).astype(jnp.uint16), bf)
            hi = lax.bitcast_convert_type(
                (u >> 16).astype(jnp.uint16), bf)
            return lo, hi

        def dot2(lohi, w_ref):
            w = w_ref[...]
            return (jnp.dot(lohi[0], w[:128], preferred_element_type=f32)
                    + jnp.dot(lohi[1], w[128:], preferred_element_type=f32))

        hd = unpack(hd_ref)
        hs = unpack(hs_ref)
        ef_b = ef_ref[...]
        q_d = dot2(hd, wq_ref) + bq_ref[...]
        k_d = dot2(hd, wk_ref) + bk_ref[...]
        k_s = dot2(hs, wk_ref) + bk_ref[...]
        v_d = dot2(hd, wv_ref) + bv_ref[...]
        v_s = dot2(hs, wv_ref) + bv_ref[...]
        e = dot16(ef_b, we_ref) + be_ref[...]
        alpha = jnp.concatenate([q_d * k_d, q_d * k_s, q_d * e], axis=1) * inv_s
        mu = jnp.mean(alpha, axis=1, keepdims=True)
        var = jnp.mean(alpha * alpha, axis=1, keepdims=True) - mu * mu
        gate = jax.nn.sigmoid((alpha - mu) * lax.rsqrt(var + 1e-5)
                              * g1_ref[...] + b1_ref[...])
        m = (dot16(v_d, wm1_ref) + dot16(v_s, wm2_ref) + dot16(e, wm3_ref)
             + bm_ref[...])
        m = m * gate
        msg = dot16(m, wmsg_ref) + bmsg_ref[...]
        mu2 = jnp.mean(msg, axis=1, keepdims=True)
        var2 = jnp.mean(msg * msg, axis=1, keepdims=True) - mu2 * mu2
        msg = (msg - mu2) * lax.rsqrt(var2 + 1e-5) * g2_ref[...] + b2_ref[...]
        eid = ((pl.program_id(0) + blk0) * EBLK
               + lax.broadcasted_iota(jnp.int32, (EBLK, 1), 0))
        msg = jnp.where(eid < n_real, msg, 0.0)
        m0_ref[...] = msg[:, :128]
        m1_ref[...] = msg[:, 128:]

    w_spec = lambda shape: pl.BlockSpec(shape, lambda i: (0, 0))
    out = pl.pallas_call(
        body,
        grid=(grid,),
        in_specs=[
            pl.BlockSpec((EBLK, 128), lambda i: (i, 0)),
            pl.BlockSpec((EBLK, 128), lambda i: (i + nb, 0)),
            pl.BlockSpec((EBLK, C), lambda i: (i + blk0, 0)),
            w_spec((C, C)), w_spec((C, C)), w_spec((C, C)), w_spec((C, C)),
            w_spec((1, C)), w_spec((1, C)), w_spec((1, C)), w_spec((1, C)),
            w_spec((C, 3 * C)), w_spec((C, 3 * C)), w_spec((C, 3 * C)),
            w_spec((1, 3 * C)), w_spec((1, 3 * C)), w_spec((1, 3 * C)),
            w_spec((3 * C, C)), w_spec((1, C)), w_spec((1, C)), w_spec((1, C)),
        ],
        out_specs=[
            pl.BlockSpec((EBLK, 128), lambda i: (i, 0)),
            pl.BlockSpec((EBLK, 128), lambda i: (i, 0)),
        ],
        out_shape=[
            jax.ShapeDtypeStruct((epad, 128), jnp.float32),
            jax.ShapeDtypeStruct((epad, 128), jnp.float32),
        ],
    )
    bf = jnp.bfloat16
    out = out(hds, hds, ef,
              p["Wq"].astype(bf), p["Wk"].astype(bf), p["Wv"].astype(bf),
              p["We"].astype(bf),
              p["bq"].reshape(1, C), p["bk"].reshape(1, C),
              p["bv"].reshape(1, C), p["be"].reshape(1, C),
              p["Wm"][:C].astype(bf), p["Wm"][C:2 * C].astype(bf),
              p["Wm"][2 * C:].astype(bf),
              p["bm"].reshape(1, 3 * C), p["g1"].reshape(1, 3 * C),
              p["b1"].reshape(1, 3 * C),
              p["Wmsg"].astype(bf), p["bmsg"].reshape(1, C),
              p["g2"].reshape(1, C), p["b2"].reshape(1, C))
    return out


def _conv_node_call(agg0, agg1, h, p):
    n = h.shape[0]

    def body(a0_ref, a1_ref, h_ref, wc0_ref, wc1_ref, bc_ref, g_ref, b_ref,
             out_ref):
        out = a0_ref[...] @ wc0_ref[...] + a1_ref[...] @ wc1_ref[...] + bc_ref[...]
        mu = jnp.mean(out, axis=0, keepdims=True)
        var = jnp.mean(out * out, axis=0, keepdims=True) - mu * mu
        bn = (out - mu) * lax.rsqrt(var + 1e-5) * g_ref[...] + b_ref[...]
        out_ref[...] = jax.nn.softplus(h_ref[...] + bn)

    return pl.pallas_call(
        body,
        grid=(2,),
        in_specs=[
            pl.BlockSpec((n, 128), lambda j: (0, 0)),
            pl.BlockSpec((n, 128), lambda j: (0, 0)),
            pl.BlockSpec((n, 128), lambda j: (0, j)),
            pl.BlockSpec((128, 128), lambda j: (0, j)),
            pl.BlockSpec((128, 128), lambda j: (0, j)),
            pl.BlockSpec((1, 128), lambda j: (0, j)),
            pl.BlockSpec((1, 128), lambda j: (0, j)),
            pl.BlockSpec((1, 128), lambda j: (0, j)),
        ],
        out_specs=pl.BlockSpec((n, 128), lambda j: (0, j)),
        out_shape=jax.ShapeDtypeStruct((n, C), jnp.float32),
    )(agg0, agg1, h, p["Wc"][:128], p["Wc"][128:], p["bc"].reshape(1, C),
      p["gbn"].reshape(1, C), p["bbn"].reshape(1, C))


def _hn_call(h, wn_pad):
    n = h.shape[0]

    def body(h_ref, w_ref, out_ref):
        out_ref[...] = h_ref[...] @ w_ref[...]

    return pl.pallas_call(
        body,
        out_shape=jax.ShapeDtypeStruct((n, 128), jnp.float32),
    )(h, wn_pad)


def _equi_edge_call(hns, ef, ea8, wes, bes, n_real):
    epad = ef.shape[0]
    grid = epad // EBLK
    c0, c1, c2, c3, c4 = (0.28209479177, 0.48860251190, 1.09254843059,
                          0.31539156525, 0.54627421529)

    def body(hn_ref, ef_ref, ea_ref, wes_ref, bes_ref, *out_refs):
        ea = ea_ref[...]
        vx, vy, vz = ea[:, 0:1], ea[:, 1:2], ea[:, 2:3]
        nrm = jnp.sqrt(vx * vx + vy * vy + vz * vz) + 1e-8
        x = vx / nrm
        y = vy / nrm
        z = vz / nrm
        es = jax.nn.silu(ef_ref[...] @ wes_ref[...] + bes_ref[...])
        m = hn_ref[...][:, :32] * es
        eid = pl.program_id(0) * EBLK + lax.broadcasted_iota(jnp.int32, (EBLK, 1), 0)
        m = jnp.where(eid < n_real, m, 0.0)
        sh = [jnp.full_like(x, c0), c1 * y, c1 * z, c1 * x,
              c2 * x * y, c2 * y * z, c3 * (3.0 * z * z - 1.0),
              c2 * x * z, c4 * (x * x - y * y)]
        out_refs[0][...] = jnp.concatenate([m * sh[j] for j in range(4)], axis=1)
        out_refs[1][...] = jnp.concatenate([m * sh[j] for j in range(4, 8)], axis=1)
        out_refs[2][...] = jnp.concatenate(
            [m * sh[8], jnp.zeros((EBLK, 96), jnp.float32)], axis=1)

    return pl.pallas_call(
        body,
        grid=(grid,),
        in_specs=[
            pl.BlockSpec((EBLK, 128), lambda i: (i, 0)),
            pl.BlockSpec((EBLK, C), lambda i: (i, 0)),
            pl.BlockSpec((EBLK, 8), lambda i: (i, 0)),
            pl.BlockSpec((C, 32), lambda i: (0, 0)),
            pl.BlockSpec((1, 32), lambda i: (0, 0)),
        ],
        out_specs=[pl.BlockSpec((EBLK, 128), lambda i: (i, 0))] * 3,
        out_shape=[jax.ShapeDtypeStruct((epad, 128), jnp.float32)] * 3,
    )(hns, ef, ea8, wes, bes)


def _equi_node_call(h, aggs, wtps, gln, bln):
    n = h.shape[0]

    def body(*refs):
        h_ref = refs[0]
        agg_refs = refs[1:4]
        wtp_refs = refs[4:7]
        g_ref, b_ref = refs[7], refs[8]
        out_ref = refs[9]
        acc = agg_refs[0][...] @ wtp_refs[0][...]
        for j in range(1, 3):
            acc = acc + agg_refs[j][...] @ wtp_refs[j][...]
        mu = jnp.mean(acc, axis=1, keepdims=True)
        var = jnp.mean(acc * acc, axis=1, keepdims=True) - mu * mu
        ln = (acc - mu) * lax.rsqrt(var + 1e-5) * g_ref[...] + b_ref[...]
        out_ref[...] = h_ref[...] + jax.nn.silu(ln)

    rblk = 1000 if n % 1000 == 0 else n
    return pl.pallas_call(
        body,
        grid=(n // rblk,),
        in_specs=[pl.BlockSpec((rblk, C), lambda i: (i, 0))]
        + [pl.BlockSpec((rblk, 128), lambda i: (i, 0))] * 3
        + [pl.BlockSpec((128, C), lambda i: (0, 0))] * 3
        + [pl.BlockSpec((1, C), lambda i: (0, 0))] * 2,
        out_specs=pl.BlockSpec((rblk, C), lambda i: (i, 0)),
        out_shape=jax.ShapeDtypeStruct((n, C), jnp.float32),
    )(h, *aggs, *wtps, gln.reshape(1, C), bln.reshape(1, C))


def _pool_call(h, batch8, wfc, bfc, wout_pad, bout_pad, ng):
    n = h.shape[0]

    def body(h_ref, b_ref, wfc_ref, bfc_ref, wo_ref, bo_ref, out_ref):
        ids = b_ref[...][:, 0:1]
        oh = (ids == lax.broadcasted_iota(jnp.int32, (1, ng), 1)).astype(jnp.float32)
        pooled = lax.dot_general(oh, h_ref[...], (((0,), (0,)), ((), ())))
        ones = jnp.ones((n, 1), jnp.float32)
        cnt = lax.dot_general(oh, ones, (((0,), (0,)), ((), ())))
        pooled = pooled / jnp.maximum(cnt, 1.0)
        hh = jax.nn.silu(pooled @ wfc_ref[...] + bfc_ref[...])
        logits = hh @ wo_ref[...] + bo_ref[...]
        l4 = logits[:, 0:4]
        mx = jnp.max(l4, axis=1, keepdims=True)
        lse = jnp.log(jnp.sum(jnp.exp(l4 - mx), axis=1, keepdims=True))
        res = l4 - mx - lse
        pad = jnp.zeros((ng, 124), jnp.float32)
        out_ref[...] = jnp.concatenate([res, pad], axis=1)

    return pl.pallas_call(
        body,
        out_shape=jax.ShapeDtypeStruct((ng, 128), jnp.float32),
    )(h, batch8, wfc, bfc, wout_pad, bout_pad)


def _gather_rows(table, idx, width):
    n_rows = idx.shape[0]
    per_tile = n_rows // NW
    n_chunks = per_tile // GCHUNK
    mesh = plsc.VectorSubcoreMesh(core_axis_name="c", subcore_axis_name="s")

    nb = 6 if width <= 128 else 3
    assert n_chunks % nb == 0

    @functools.partial(
        pl.kernel,
        mesh=mesh,
        out_type=jax.ShapeDtypeStruct((n_rows, width), jnp.float32),
        scratch_types=[
            pltpu.VMEM((per_tile,), jnp.int32),
        ] + [pltpu.VMEM((GCHUNK, width), jnp.float32) for _ in range(nb)]
        + [pltpu.SemaphoreType.DMA for _ in range(2 * nb)],
    )
    def k(table_hbm, idx_hbm, out_hbm, idx_all, *rest):
        bufs = rest[:nb]
        sg = rest[nb:2 * nb]
        sw = rest[2 * nb:3 * nb]
        cid = lax.axis_index("c")
        sid = lax.axis_index("s")
        base = pl.multiple_of((sid * NC + cid) * per_tile, GCHUNK)
        pltpu.sync_copy(idx_hbm.at[pl.ds(base, per_tile)], idx_all)

        def gsrc(j):
            o = pl.multiple_of(j * GCHUNK, GCHUNK)
            return table_hbm.at[idx_all.at[pl.ds(o, GCHUNK)]]

        def wdst(j):
            r = pl.multiple_of(base + j * GCHUNK, GCHUNK)
            return out_hbm.at[pl.ds(r, GCHUNK)]

        for b in range(nb):
            pltpu.async_copy(gsrc(b), bufs[b], sg[b])

        def body(jj, carry):
            j0 = jj * nb
            for b in range(nb):
                pltpu.make_async_copy(gsrc(j0 + b), bufs[b], sg[b]).wait()
                pltpu.async_copy(bufs[b], wdst(j0 + b), sw[b])
            for b in range(nb):
                pltpu.make_async_copy(bufs[b], wdst(j0 + b), sw[b]).wait()

                @pl.when(jj < n_chunks // nb - 1)
                def _(b=b):
                    pltpu.async_copy(gsrc(j0 + nb + b), bufs[b], sg[b])

            return carry

        lax.fori_loop(0, n_chunks // nb, body, 0)

    return k(table, idx)


def _scatter_add(msgs, dst, n_nodes, core_of):
    msgs = [m if isinstance(m, tuple) else (m,) for m in msgs]
    nh = len(msgs[0])
    assert all(len(m) == nh for m in msgs) and NW % nh == 0
    tiles_per_half = NW // nh
    epad = dst.shape[0]
    per_tile = epad // NW
    n_chunks = per_tile // GCHUNK
    stripe = (n_nodes // NS) // 8 * 8
    last_stripe = n_nodes - (NS - 1) * stripe
    na = len(msgs)
    widths = [m[0].shape[1] for m in msgs]
    w = widths[0]
    assert all(wi == w for wi in widths)
    groups = [[a for a in range(na) if core_of[a] == c] for c in range(NC)]
    assert max(len(g) for g in groups) == 1
    mesh = plsc.VectorSubcoreMesh(core_axis_name="c", subcore_axis_name="s")

    zeros = jnp.zeros((last_stripe, w), jnp.float32)
    dst3 = dst.reshape(NW, n_chunks, GCHUNK)

    nb = 2
    assert n_chunks % nb == 0
    scratch = (
        [pltpu.VMEM((n_chunks, GCHUNK), jnp.int32)]
        + [pltpu.VMEM((GCHUNK, w), jnp.float32) for _ in range(nb)]
        + [pltpu.VMEM_SHARED((n_nodes, w), jnp.float32)]
        + [pltpu.SemaphoreType.DMA for _ in range(nb)]
    )

    @functools.partial(
        pl.kernel,
        mesh=mesh,
        out_type=[jax.ShapeDtypeStruct((n_nodes, w), jnp.float32)
                  for _ in range(na)],
        scratch_types=scratch,
    )
    def k(*refs):
        nm = na * nh
        msg_refs = refs[0:nm]
        dst_ref = refs[nm]
        zero_ref = refs[nm + 1]
        out_refs = refs[nm + 2:nm + na + 2]
        rest = refs[nm + na + 2:]
        idx2d = rest[0]
        bufs = rest[1:1 + nb]
        acc = rest[1 + nb]
        sl = rest[2 + nb:2 + 2 * nb]

        cid = lax.axis_index("c")
        sid = lax.axis_index("s")
        wid = sid * NC + cid
        base = pl.multiple_of(wid * per_tile, GCHUNK)
        off = pl.multiple_of(sid * stripe, 8)

        for length, pred in ((stripe, sid < NS - 1),
                             (last_stripe, sid == NS - 1)):

            @pl.when(pred)
            def _(length=length):
                pltpu.sync_copy(zero_ref.at[pl.ds(0, length)],
                                acc.at[pl.ds(off, length)])

        plsc.subcore_barrier()
        pltpu.sync_copy(dst_ref.at[wid], idx2d)

        def accum(mref, mbase):
            def lsrc(j):
                r = pl.multiple_of(mbase + j * GCHUNK, GCHUNK)
                return mref.at[pl.ds(r, GCHUNK)]

            for b in range(nb):
                pltpu.async_copy(lsrc(b), bufs[b], sl[b])

            def body(jj, carry):
                j0 = jj * nb
                for b in range(nb):
                    pltpu.make_async_copy(lsrc(j0 + b), bufs[b], sl[b]).wait()
                    pltpu.sync_copy(bufs[b], acc.at[idx2d.at[j0 + b]],
                                    add=True)

                    @pl.when(jj < n_chunks // nb - 1)
                    def _(b=b):
                        pltpu.async_copy(lsrc(j0 + nb + b), bufs[b], sl[b])

                return carry

            lax.fori_loop(0, n_chunks // nb, body, 0)

        for c in range(NC):
            if not groups[c]:
                continue
            a = groups[c][0]
            for hh in range(nh):
                lo = hh * tiles_per_half
                pred = jnp.logical_and(
                    cid == c,
                    jnp.logical_and(wid >= lo, wid < lo + tiles_per_half))

                @pl.when(pred)
                def _(a=a, hh=hh, lo=lo):
                    mbase = pl.multiple_of((wid - lo) * per_tile, GCHUNK)
                    accum(msg_refs[a * nh + hh], mbase)

        plsc.subcore_barrier()

        for c in range(NC):
            if not groups[c]:
                continue
            a = groups[c][0]
            for length, pred in ((stripe, sid < NS - 1),
                                 (last_stripe, sid == NS - 1)):

                @pl.when(jnp.logical_and(cid == c, pred))
                def _(a=a, length=length):
                    pltpu.sync_copy(acc.at[pl.ds(off, length)],
                                    out_refs[a].at[pl.ds(off, length)])

    return k(*[m for t in msgs for m in t], dst3, zeros)


def kernel(x, edge_attr, inv_edge_attr, params, wyckoff, edge_index,
           inv_edge_index, inv_edge_type, batch):
    n = x.shape[0]
    e_r = edge_attr.shape[0]
    e_i = inv_edge_attr.shape[0]
    ne = e_r + e_i
    epad = _cdiv(ne, NW * GCHUNK * 2) * NW * GCHUNK * 2
    ng = 64
    f32 = jnp.float32

    ea8 = jnp.concatenate([
        jnp.concatenate([edge_attr, jnp.zeros((e_r, 5), f32)], axis=1),
        jnp.concatenate([inv_edge_attr, jnp.ones((e_i, 1), f32),
                         inv_edge_type.astype(f32)[:, None],
                         jnp.zeros((e_i, 3), f32)], axis=1),
        jnp.concatenate([jnp.ones((epad - ne, 1), f32),
                         jnp.zeros((epad - ne, 7), f32)], axis=1),
    ], axis=0)

    zpad = jnp.zeros((epad - ne,), jnp.int32)
    src = jnp.concatenate([edge_index[0], inv_edge_index[0], zpad])
    dst = jnp.concatenate([edge_index[1], inv_edge_index[1], zpad])
    ehalf = epad // 2
    idx_a = jnp.concatenate([dst[:ehalf], src[:ehalf]])
    idx_b = jnp.concatenate([dst[ehalf:], src[ehalf:]])

    x_pad = jnp.concatenate([x, jnp.zeros((n, 128 - x.shape[1]), f32)], axis=1)
    wyk8 = jnp.broadcast_to(wyckoff.astype(jnp.int32)[:, None], (n, 8))
    batch8 = jnp.broadcast_to(batch.astype(jnp.int32)[:, None], (n, 8))

    wa1 = jnp.concatenate([params["W_atom"][:x.shape[1]],
                           jnp.zeros((128 - x.shape[1], C), f32)], axis=0)
    t2 = params["wyckoff_table"] @ params["W_atom"][x.shape[1]:]
    t2 = jnp.concatenate([t2, jnp.zeros((128 - t2.shape[0], C), f32)], axis=0)

    w2 = params["W_rbf"][C:]
    u_vec = (params["W_inv"] @ w2).reshape(1, C)
    w_vec = (params["b_inv"] @ w2).reshape(1, C)

    equi = params["equi"]
    wtp = equi["Wtp"]
    wtp_g = [
        jnp.concatenate([wtp[j::9] for j in range(4)], axis=0),
        jnp.concatenate([wtp[j::9] for j in range(4, 8)], axis=0),
        jnp.concatenate([wtp[8::9], jnp.zeros((96, C), f32)], axis=0),
    ]
    wn_pad = jnp.concatenate([equi["Wn"], jnp.zeros((C, 96), f32)], axis=1)

    wout_pad = jnp.concatenate([params["W_out"],
                                jnp.zeros((C, 124), f32)], axis=1)
    bout_pad = jnp.concatenate([params["b_out"],
                                jnp.zeros((124,), f32)]).reshape(1, 128)

    h = _embed_call(x_pad, wyk8, wa1, t2, params["b_atom"].reshape(1, C))
    ef = _ef_call(ea8, params["W_rbf"][:C], u_vec, w_vec,
                  params["b_rbf"].reshape(1, C), ne)

    def conv(h, p):
        hp = _pack_call(h)
        hds_a = _gather_rows(hp, idx_a, 128)
        hds_b = _gather_rows(hp, idx_b, 128)
        ma0, ma1 = _conv_edge_call(hds_a, ef, p, ne, 0)
        mb0, mb1 = _conv_edge_call(hds_b, ef, p, ne, ehalf // EBLK)
        agg0, agg1 = _scatter_add([(ma0, mb0), (ma1, mb1)], dst, n, [0, 1])
        return _conv_node_call(agg0, agg1, h, p)

    h = conv(h, params["conv0"])

    hn = _hn_call(h, wn_pad)
    hns = _gather_rows(hn, src, 128)
    tps = _equi_edge_call(hns, ef, ea8, equi["Wes"],
                          equi["bes"].reshape(1, 32), ne)
    aggs01 = _scatter_add(tps[:2], dst, n, [0, 1])
    aggs2 = _scatter_add(tps[2:], dst, n, [0])
    h = _equi_node_call(h, aggs01 + aggs2, wtp_g, equi["gln"], equi["bln"])

    h = conv(h, params["conv1"])
    h = conv(h, params["conv2"])

    out = _pool_call(h, batch8, params["W_fc"], params["b_fc"].reshape(1, C),
                     wout_pad, bout_pad, ng)
    return out[:, :4]
